# Initial kernel scaffold; baseline (speedup 1.0000x reference)
import jax
import jax.numpy as jnp
from jax.experimental import pallas as pl


def kernel(atom_features, bond_features, kernel, bias_p, w_ih, w_hh, b_ih, b_hh, ipw, ipb, opw, opb, ln1_g, ln1_b, w1, b1, w2, b2, ln2_g, ln2_b, d1_w, d1_b, d2_w, d2_b, pair_indices, molecule_indicator):
    raise NotImplementedError("write your pallas kernel here")


# speedup vs baseline: 1.7766x; 1.7766x over previous
import functools

import jax
import jax.numpy as jnp
from jax import lax
from jax.experimental import pallas as pl
from jax.experimental.pallas import tpu as pltpu
from jax.experimental.pallas import tpu_sc as plsc

_E = 16384
_N = 4096
_U = 64
_B = 32
_L = 512
_NC, _NS = 2, 16
_NW = _NC * _NS
_EPW = _E // _NW
_CHUNK = 128
_NCH = _EPW // _CHUNK
_RPS = _N // _NS

_SC_MESH = plsc.VectorSubcoreMesh(core_axis_name="c", subcore_axis_name="s")
_SC_PARAMS = pltpu.CompilerParams(use_tc_tiling_on_sc=False)


def _gather_body(table_hbm, idx_hbm, out_hbm, idx_v, rows_v, sem):
    wid = lax.axis_index("s") * _NC + lax.axis_index("c")
    base = wid * _EPW
    pltpu.sync_copy(idx_hbm.at[pl.ds(base, _EPW)], idx_v)
    descs = [
        pltpu.async_copy(
            table_hbm.at[idx_v.at[pl.ds(j * _CHUNK, _CHUNK)]],
            rows_v.at[pl.ds(j * _CHUNK, _CHUNK)], sem)
        for j in range(_NCH)
    ]
    for d in descs:
        d.wait()
    pltpu.sync_copy(rows_v, out_hbm.at[pl.ds(base, _EPW)])


def _sc_gather(table, idx):
    return pl.kernel(
        _gather_body,
        out_type=jax.ShapeDtypeStruct((_E, _U), jnp.float32),
        mesh=_SC_MESH,
        scratch_types=[
            pltpu.VMEM((_EPW,), jnp.int32),
            pltpu.VMEM((_EPW, _U), jnp.float32),
            pltpu.SemaphoreType.DMA,
        ],
        compiler_params=_SC_PARAMS,
    )(table, idx)


def _scatter_body(msg_hbm, dst_hbm, zero_hbm, out_hbm, idx_v, msg_v, agg_sh):
    cid = lax.axis_index("c")
    sid = lax.axis_index("s")
    wid = sid * _NC + cid
    rows0 = sid * _RPS
    pltpu.sync_copy(zero_hbm.at[pl.ds(rows0, _RPS)],
                    agg_sh.at[pl.ds(rows0, _RPS)])
    plsc.subcore_barrier()
    base = wid * _EPW
    pltpu.sync_copy(msg_hbm.at[pl.ds(base, _EPW)], msg_v)
    pltpu.sync_copy(dst_hbm.at[pl.ds(wid * _NCH, _NCH)], idx_v)
    for j in range(_NCH):
        pltpu.sync_copy(msg_v.at[pl.ds(j * _CHUNK, _CHUNK)],
                        agg_sh.at[idx_v.at[j]], add=True)
    plsc.subcore_barrier()
    pltpu.sync_copy(agg_sh.at[pl.ds(rows0, _RPS)],
                    out_hbm.at[pl.ds(cid * _N + rows0, _RPS)])


def _sc_scatter(msg, dst2d, zeros_nu):
    return pl.kernel(
        _scatter_body,
        out_type=jax.ShapeDtypeStruct((_NC * _N, _U), jnp.float32),
        mesh=_SC_MESH,
        scratch_types=[
            pltpu.VMEM((_NCH, _CHUNK), jnp.int32),
            pltpu.VMEM((_EPW, _U), jnp.float32),
            pltpu.VMEM_SHARED((_N, _U), jnp.float32),
        ],
        compiler_params=_SC_PARAMS,
    )(msg, dst2d, zeros_nu)


def _plan_body(mol_ref, gidx_ref, valid_ref, counts_ref):
    mol = mol_ref[...]
    mids = lax.broadcasted_iota(jnp.int32, (_B, _N), 0)
    counts = jnp.sum((mol == mids).astype(jnp.int32), axis=1, keepdims=True)
    starts = jnp.sum((mids > mol).astype(jnp.int32), axis=1, keepdims=True)
    p = lax.broadcasted_iota(jnp.int32, (_B, _L), 1)
    lim = jnp.minimum(counts, _L)
    valid_ref[...] = (p < counts).astype(jnp.float32)
    gidx_ref[...] = jnp.where(p < lim, starts + p, 0)
    counts_ref[...] = counts


def _plan(mol2d):
    return pl.pallas_call(
        _plan_body,
        out_shape=(
            jax.ShapeDtypeStruct((_B, _L), jnp.int32),
            jax.ShapeDtypeStruct((_B, _L), jnp.float32),
            jax.ShapeDtypeStruct((_B, 1), jnp.int32),
        ),
    )(mol2d)


def _msg_body(neigh_ref, bfa_ref, kt_ref, out_ref):
    p = jnp.dot(neigh_ref[...], kt_ref[...],
                preferred_element_type=jnp.float32)
    acc = bfa_ref[:, 0:1] * p[:, 0:_U]
    for b in range(1, 17):
        acc = acc + bfa_ref[:, b:b + 1] * p[:, b * _U:(b + 1) * _U]
    out_ref[...] = acc


def _msg(neigh, bfa, kt2):
    blk = 2048
    return pl.pallas_call(
        _msg_body,
        grid=(_E // blk,),
        in_specs=[
            pl.BlockSpec((blk, _U), lambda i: (i, 0)),
            pl.BlockSpec((blk, 17), lambda i: (i, 0)),
            pl.BlockSpec((_U, 17 * _U), lambda i: (0, 0)),
        ],
        out_specs=pl.BlockSpec((blk, _U), lambda i: (i, 0)),
        out_shape=jax.ShapeDtypeStruct((_E, _U), jnp.float32),
    )(neigh, bfa, kt2)


def _gru_body(p0_ref, p1_ref, h_ref, wih_ref, whh_ref, bih_ref, bhh_ref,
              out_ref):
    agg = p0_ref[...] + p1_ref[...]
    gi = jnp.dot(agg, wih_ref[...],
                 preferred_element_type=jnp.float32) + bih_ref[...]
    gh = jnp.dot(h_ref[...], whh_ref[...],
                 preferred_element_type=jnp.float32) + bhh_ref[...]
    r = jax.nn.sigmoid(gi[:, 0:_U] + gh[:, 0:_U])
    z = jax.nn.sigmoid(gi[:, _U:2 * _U] + gh[:, _U:2 * _U])
    n = jnp.tanh(gi[:, 2 * _U:3 * _U] + r * gh[:, 2 * _U:3 * _U])
    out_ref[...] = (1.0 - z) * n + z * h_ref[...]


def _gru(partials, h, wih_t, whh_t, bih2, bhh2):
    blk = 1024
    nb = _N // blk
    return pl.pallas_call(
        _gru_body,
        grid=(nb,),
        in_specs=[
            pl.BlockSpec((blk, _U), lambda i: (i, 0)),
            pl.BlockSpec((blk, _U), lambda i, _nb=nb: (i + _nb, 0)),
            pl.BlockSpec((blk, _U), lambda i: (i, 0)),
            pl.BlockSpec((_U, 3 * _U), lambda i: (0, 0)),
            pl.BlockSpec((_U, 3 * _U), lambda i: (0, 0)),
            pl.BlockSpec((1, 3 * _U), lambda i: (0, 0)),
            pl.BlockSpec((1, 3 * _U), lambda i: (0, 0)),
        ],
        out_specs=pl.BlockSpec((blk, _U), lambda i: (i, 0)),
        out_shape=jax.ShapeDtypeStruct((_N, _U), jnp.float32),
    )(partials, partials, h, wih_t, whh_t, bih2, bhh2)


def _attn_body(x_ref, vr_ref, vc_ref, ipw_ref, ipb_ref, opw_ref, opb_ref,
               ln1g_ref, ln1b_ref, w1_ref, b1_ref, w2_ref, b2_ref,
               ln2g_ref, ln2b_ref, vs_ref, pr_ref):
    hd = _U // 8
    x = x_ref[0] * vc_ref[0]
    keymask = vr_ref[0]
    qkv = jnp.dot(x, ipw_ref[...],
                  preferred_element_type=jnp.float32) + ipb_ref[...]
    q = qkv[:, 0:_U]
    k = qkv[:, _U:2 * _U]
    v = qkv[:, 2 * _U:3 * _U]
    scale = 1.0 / jnp.sqrt(jnp.float32(hd))
    ctx_parts = []
    for h in range(8):
        qh = q[:, h * hd:(h + 1) * hd]
        kh = k[:, h * hd:(h + 1) * hd]
        vh = v[:, h * hd:(h + 1) * hd]
        s = lax.dot_general(qh, kh, (((1,), (1,)), ((), ())),
                            preferred_element_type=jnp.float32) * scale
        s = jnp.where(keymask > 0, s, jnp.float32(-1e9))
        m = jnp.max(s, axis=1, keepdims=True)
        e = jnp.exp(s - m)
        a = e / jnp.sum(e, axis=1, keepdims=True)
        ctx_parts.append(jnp.dot(a, vh, preferred_element_type=jnp.float32))
    ctx = jnp.concatenate(ctx_parts, axis=1)
    attn_out = jnp.dot(ctx, opw_ref[...],
                       preferred_element_type=jnp.float32) + opb_ref[...]
    y = x + attn_out
    mu = jnp.mean(y, axis=1, keepdims=True)
    var = jnp.mean((y - mu) ** 2, axis=1, keepdims=True)
    pin = (y - mu) / jnp.sqrt(var + 1e-5) * ln1g_ref[...] + ln1b_ref[...]
    hid = jnp.maximum(
        jnp.dot(pin, w1_ref[...], preferred_element_type=jnp.float32)
        + b1_ref[...], 0.0)
    mlp = jnp.dot(hid, w2_ref[...],
                  preferred_element_type=jnp.float32) + b2_ref[...]
    y2 = pin + mlp
    mu2 = jnp.mean(y2, axis=1, keepdims=True)
    var2 = jnp.mean((y2 - mu2) ** 2, axis=1, keepdims=True)
    pout = (y2 - mu2) / jnp.sqrt(var2 + 1e-5) * ln2g_ref[...] + ln2b_ref[...]
    vs_ref[0] = jnp.sum(pout * vc_ref[0], axis=0, keepdims=True)
    pr_ref[0] = pout[_L - 1:_L, :]


def _attn(x3, vr, vc, ipw_t, ipb2, opw_t, opb2, ln1g2, ln1b2, w1_t, b12,
          w2_t, b22, ln2g2, ln2b2):
    full = lambda a, b: pl.BlockSpec((a, b), lambda i: (0, 0))
    return pl.pallas_call(
        _attn_body,
        grid=(_B,),
        in_specs=[
            pl.BlockSpec((1, _L, _U), lambda i: (i, 0, 0)),
            pl.BlockSpec((1, 1, _L), lambda i: (i, 0, 0)),
            pl.BlockSpec((1, _L, 1), lambda i: (i, 0, 0)),
            full(_U, 3 * _U), full(1, 3 * _U),
            full(_U, _U), full(1, _U),
            full(1, _U), full(1, _U),
            full(_U, _U), full(1, _U),
            full(_U, _U), full(1, _U),
            full(1, _U), full(1, _U),
        ],
        out_specs=(
            pl.BlockSpec((1, 1, _U), lambda i: (i, 0, 0)),
            pl.BlockSpec((1, 1, _U), lambda i: (i, 0, 0)),
        ),
        out_shape=(
            jax.ShapeDtypeStruct((_B, 1, _U), jnp.float32),
            jax.ShapeDtypeStruct((_B, 1, _U), jnp.float32),
        ),
    )(x3, vr, vc, ipw_t, ipb2, opw_t, opb2, ln1g2, ln1b2, w1_t, b12,
      w2_t, b22, ln2g2, ln2b2)


def _final_body(vs_ref, pr_ref, cnt_ref, d1_ref, b1_ref, d2_ref, b2_ref,
                out_ref):
    cnt = cnt_ref[...].astype(jnp.float32)
    mx = jnp.max(cnt)
    pooled = (vs_ref[...] + (mx - cnt) * pr_ref[...]) / mx
    hid = jnp.maximum(
        jnp.dot(pooled, d1_ref[...], preferred_element_type=jnp.float32)
        + b1_ref[...], 0.0)
    logit = jnp.dot(hid, d2_ref[...],
                    preferred_element_type=jnp.float32) + b2_ref[...]
    out_ref[...] = jax.nn.sigmoid(logit)


def _final(vs, pr, counts, d1_t, d1b2, d2_t, d2b2):
    return pl.pallas_call(
        _final_body,
        out_shape=jax.ShapeDtypeStruct((_B, 1), jnp.float32),
    )(vs, pr, counts, d1_t, d1b2, d2_t, d2b2)


def kernel(atom_features, bond_features, kernel, bias_p, w_ih, w_hh, b_ih,
           b_hh, ipw, ipb, opw, opb, ln1_g, ln1_b, w1, b1, w2, b2, ln2_g,
           ln2_b, d1_w, d1_b, d2_w, d2_b, pair_indices, molecule_indicator):
    src = pair_indices[:, 1]
    dst2d = pair_indices[:, 0].reshape(_NW * _NCH, _CHUNK)
    bfa = jnp.concatenate(
        [bond_features, jnp.ones((_E, 1), jnp.float32)], axis=1)
    k3 = jnp.concatenate([kernel, bias_p[None, :]], axis=0)
    kt2 = k3.reshape(17, _U, _U).transpose(2, 0, 1).reshape(_U, 17 * _U)
    zeros_nu = jnp.zeros((_N, _U), jnp.float32)

    gidx, validf, counts = _plan(molecule_indicator.reshape(1, _N))

    h = atom_features
    for _ in range(4):
        neigh = _sc_gather(h, src)
        msg = _msg(neigh, bfa, kt2)
        partials = _sc_scatter(msg, dst2d, zeros_nu)
        h = _gru(partials, h, w_ih.T, w_hh.T, b_ih[None, :], b_hh[None, :])

    xg = _sc_gather(h, gidx.reshape(_E))
    vs, pr = _attn(
        xg.reshape(_B, _L, _U), validf.reshape(_B, 1, _L),
        validf.reshape(_B, _L, 1),
        ipw.T, ipb[None, :], opw.T, opb[None, :],
        ln1_g[None, :], ln1_b[None, :], w1.T, b1[None, :],
        w2.T, b2[None, :], ln2_g[None, :], ln2_b[None, :])
    return _final(vs.reshape(_B, _U), pr.reshape(_B, _U), counts, d1_w.T,
                  d1_b[None, :], d2_w.T, d2_b[None, :])


# speedup vs baseline: 2.1955x; 1.2358x over previous
import functools

import jax
import jax.numpy as jnp
from jax import lax
from jax.experimental import pallas as pl
from jax.experimental.pallas import tpu as pltpu
from jax.experimental.pallas import tpu_sc as plsc

_E = 16384
_N = 4096
_U = 64
_B = 32
_L = 512
_NC, _NS = 2, 16
_NW = _NC * _NS
_EPW = _E // _NW
_CHUNK = 128
_NCH = _EPW // _CHUNK
_RPS = _N // _NS

_SC_MESH = plsc.VectorSubcoreMesh(core_axis_name="c", subcore_axis_name="s")
_SC_PARAMS = pltpu.CompilerParams(use_tc_tiling_on_sc=False)


def _gather_body(table_hbm, idx_hbm, out_hbm, idx_v, rows_v, sem):
    wid = lax.axis_index("s") * _NC + lax.axis_index("c")
    base = wid * _EPW
    pltpu.sync_copy(idx_hbm.at[pl.ds(base, _EPW)], idx_v)
    descs = [
        pltpu.async_copy(
            table_hbm.at[idx_v.at[pl.ds(j * _CHUNK, _CHUNK)]],
            rows_v.at[pl.ds(j * _CHUNK, _CHUNK)], sem)
        for j in range(_NCH)
    ]
    for d in descs:
        d.wait()
    pltpu.sync_copy(rows_v, out_hbm.at[pl.ds(base, _EPW)])


def _sc_gather(table, idx):
    return pl.kernel(
        _gather_body,
        out_type=jax.ShapeDtypeStruct((_E, _U), jnp.float32),
        mesh=_SC_MESH,
        scratch_types=[
            pltpu.VMEM((_EPW,), jnp.int32),
            pltpu.VMEM((_EPW, _U), jnp.float32),
            pltpu.SemaphoreType.DMA,
        ],
        compiler_params=_SC_PARAMS,
    )(table, idx)


def _scatter_body(msg_hbm, dst_hbm, zero_hbm, out_hbm, idx_v, msg_v, agg_sh):
    cid = lax.axis_index("c")
    sid = lax.axis_index("s")
    wid = sid * _NC + cid
    rows0 = sid * _RPS
    pltpu.sync_copy(zero_hbm.at[pl.ds(rows0, _RPS)],
                    agg_sh.at[pl.ds(rows0, _RPS)])
    plsc.subcore_barrier()
    base = wid * _EPW
    pltpu.sync_copy(msg_hbm.at[pl.ds(base, _EPW)], msg_v)
    pltpu.sync_copy(dst_hbm.at[pl.ds(wid * _NCH, _NCH)], idx_v)
    for j in range(_NCH):
        pltpu.sync_copy(msg_v.at[pl.ds(j * _CHUNK, _CHUNK)],
                        agg_sh.at[idx_v.at[j]], add=True)
    plsc.subcore_barrier()
    pltpu.sync_copy(agg_sh.at[pl.ds(rows0, _RPS)],
                    out_hbm.at[pl.ds(cid * _N + rows0, _RPS)])


def _sc_scatter(msg, dst2d, zeros_nu):
    return pl.kernel(
        _scatter_body,
        out_type=jax.ShapeDtypeStruct((_NC * _N, _U), jnp.float32),
        mesh=_SC_MESH,
        scratch_types=[
            pltpu.VMEM((_NCH, _CHUNK), jnp.int32),
            pltpu.VMEM((_EPW, _U), jnp.float32),
            pltpu.VMEM_SHARED((_N, _U), jnp.float32),
        ],
        compiler_params=_SC_PARAMS,
    )(msg, dst2d, zeros_nu)


def _plan_body(mol_ref, gidx_ref, valid_ref, counts_ref, kb_ref):
    mol = mol_ref[...]
    mids = lax.broadcasted_iota(jnp.int32, (_B, _N), 0)
    counts = jnp.sum((mol == mids).astype(jnp.int32), axis=1, keepdims=True)
    starts = jnp.sum((mids > mol).astype(jnp.int32), axis=1, keepdims=True)
    p = lax.broadcasted_iota(jnp.int32, (_B, _L), 1)
    mrow = lax.broadcasted_iota(jnp.int32, (_B, _L), 0)
    lim = jnp.minimum(counts, _L)
    valid = (p < counts).astype(jnp.float32)
    valid_ref[...] = valid
    gidx_ref[...] = jnp.where(p < lim, starts + p,
                              jnp.bitwise_and(mrow * _L + p, _N - 1))
    counts_ref[...] = counts
    kb_ref[...] = (valid - 1.0) * jnp.float32(1e9)


def _plan(mol2d):
    return pl.pallas_call(
        _plan_body,
        out_shape=(
            jax.ShapeDtypeStruct((_B, _L), jnp.int32),
            jax.ShapeDtypeStruct((_B, _L), jnp.float32),
            jax.ShapeDtypeStruct((_B, 1), jnp.int32),
            jax.ShapeDtypeStruct((_B, _L), jnp.float32),
        ),
    )(mol2d)


def _msg_body(neigh_ref, bfa_ref, kt_ref, out_ref):
    p = jnp.dot(neigh_ref[...], kt_ref[...],
                preferred_element_type=jnp.float32)
    acc = bfa_ref[:, 0:1] * p[:, 0:_U]
    for b in range(1, 17):
        acc = acc + bfa_ref[:, b:b + 1] * p[:, b * _U:(b + 1) * _U]
    out_ref[...] = acc


def _msg(neigh, bfa, kt2):
    blk = 2048
    return pl.pallas_call(
        _msg_body,
        grid=(_E // blk,),
        in_specs=[
            pl.BlockSpec((blk, _U), lambda i: (i, 0)),
            pl.BlockSpec((blk, 17), lambda i: (i, 0)),
            pl.BlockSpec((_U, 17 * _U), lambda i: (0, 0)),
        ],
        out_specs=pl.BlockSpec((blk, _U), lambda i: (i, 0)),
        out_shape=jax.ShapeDtypeStruct((_E, _U), jnp.float32),
    )(neigh, bfa, kt2)


def _gru_body(p0_ref, p1_ref, h_ref, wih_ref, whh_ref, bih_ref, bhh_ref,
              out_ref):
    agg = p0_ref[...] + p1_ref[...]
    gi = jnp.dot(agg, wih_ref[...],
                 preferred_element_type=jnp.float32) + bih_ref[...]
    gh = jnp.dot(h_ref[...], whh_ref[...],
                 preferred_element_type=jnp.float32) + bhh_ref[...]
    r = jax.nn.sigmoid(gi[:, 0:_U] + gh[:, 0:_U])
    z = jax.nn.sigmoid(gi[:, _U:2 * _U] + gh[:, _U:2 * _U])
    n = jnp.tanh(gi[:, 2 * _U:3 * _U] + r * gh[:, 2 * _U:3 * _U])
    out_ref[...] = (1.0 - z) * n + z * h_ref[...]


def _gru(partials, h, wih_t, whh_t, bih2, bhh2):
    blk = 1024
    nb = _N // blk
    return pl.pallas_call(
        _gru_body,
        grid=(nb,),
        in_specs=[
            pl.BlockSpec((blk, _U), lambda i: (i, 0)),
            pl.BlockSpec((blk, _U), lambda i, _nb=nb: (i + _nb, 0)),
            pl.BlockSpec((blk, _U), lambda i: (i, 0)),
            pl.BlockSpec((_U, 3 * _U), lambda i: (0, 0)),
            pl.BlockSpec((_U, 3 * _U), lambda i: (0, 0)),
            pl.BlockSpec((1, 3 * _U), lambda i: (0, 0)),
            pl.BlockSpec((1, 3 * _U), lambda i: (0, 0)),
        ],
        out_specs=pl.BlockSpec((blk, _U), lambda i: (i, 0)),
        out_shape=jax.ShapeDtypeStruct((_N, _U), jnp.float32),
    )(partials, partials, h, wih_t, whh_t, bih2, bhh2)


_RT = 128
_PADR = 8


def _attn_body(cnt_ref, x_ref, kb_ref, vc_ref, ipw_ref, ipb_ref, opw_ref,
               opb_ref, ln1g_ref, ln1b_ref, w1_ref, b1_ref, w2_ref, b2_ref,
               ln2g_ref, ln2b_ref, vs_ref, pr_ref):
    hd = _U // 8
    count = cnt_ref[0, 0, 0]
    x = x_ref[0] * vc_ref[0]
    kb = kb_ref[0]
    qkv = jnp.dot(x, ipw_ref[...],
                  preferred_element_type=jnp.float32) + ipb_ref[...]
    scale = 1.0 / jnp.sqrt(jnp.float32(hd))
    q = qkv[:, 0:_U] * scale
    k = qkv[:, _U:2 * _U]
    v = qkv[:, 2 * _U:3 * _U]

    def tile(r0, rows):
        qt = q[r0:r0 + rows, :]
        ctx_parts = []
        for h in range(8):
            qh = qt[:, h * hd:(h + 1) * hd]
            kh = k[:, h * hd:(h + 1) * hd]
            vh = v[:, h * hd:(h + 1) * hd]
            s = lax.dot_general(qh, kh, (((1,), (1,)), ((), ())),
                                preferred_element_type=jnp.float32) + kb
            m = jnp.max(s, axis=1, keepdims=True)
            e = jnp.exp(s - m)
            a = e / jnp.sum(e, axis=1, keepdims=True)
            ctx_parts.append(
                jnp.dot(a, vh, preferred_element_type=jnp.float32))
        ctx = jnp.concatenate(ctx_parts, axis=1)
        attn_out = jnp.dot(ctx, opw_ref[...],
                           preferred_element_type=jnp.float32) + opb_ref[...]
        y = x[r0:r0 + rows, :] + attn_out
        mu = jnp.mean(y, axis=1, keepdims=True)
        var = jnp.mean((y - mu) ** 2, axis=1, keepdims=True)
        pin = ((y - mu) / jnp.sqrt(var + 1e-5) * ln1g_ref[...]
               + ln1b_ref[...])
        hid = jnp.maximum(
            jnp.dot(pin, w1_ref[...], preferred_element_type=jnp.float32)
            + b1_ref[...], 0.0)
        mlp = jnp.dot(hid, w2_ref[...],
                      preferred_element_type=jnp.float32) + b2_ref[...]
        y2 = pin + mlp
        mu2 = jnp.mean(y2, axis=1, keepdims=True)
        var2 = jnp.mean((y2 - mu2) ** 2, axis=1, keepdims=True)
        return ((y2 - mu2) / jnp.sqrt(var2 + 1e-5) * ln2g_ref[...]
                + ln2b_ref[...])

    vs_ref[0] = jnp.zeros((1, _U), jnp.float32)
    for r in range(_L // _RT):
        @pl.when(count > r * _RT)
        def _():
            pout = tile(r * _RT, _RT)
            vm = vc_ref[0][r * _RT:(r + 1) * _RT, :]
            vs_ref[0] = vs_ref[0] + jnp.sum(pout * vm, axis=0, keepdims=True)
    pout_pad = tile(_L - _PADR, _PADR)
    pr_ref[0] = pout_pad[_PADR - 1:_PADR, :]


def _attn(counts2, x3, kb3, vc, ipw_t, ipb2, opw_t, opb2, ln1g2, ln1b2,
          w1_t, b12, w2_t, b22, ln2g2, ln2b2):
    full = lambda a, b: pl.BlockSpec((a, b), lambda i: (0, 0))
    return pl.pallas_call(
        _attn_body,
        grid=(_B,),
        in_specs=[
            pl.BlockSpec((1, 1, 1), lambda i: (i, 0, 0),
                         memory_space=pltpu.SMEM),
            pl.BlockSpec((1, _L, _U), lambda i: (i, 0, 0)),
            pl.BlockSpec((1, 1, _L), lambda i: (i, 0, 0)),
            pl.BlockSpec((1, _L, 1), lambda i: (i, 0, 0)),
            full(_U, 3 * _U), full(1, 3 * _U),
            full(_U, _U), full(1, _U),
            full(1, _U), full(1, _U),
            full(_U, _U), full(1, _U),
            full(_U, _U), full(1, _U),
            full(1, _U), full(1, _U),
        ],
        out_specs=(
            pl.BlockSpec((1, 1, _U), lambda i: (i, 0, 0)),
            pl.BlockSpec((1, 1, _U), lambda i: (i, 0, 0)),
        ),
        out_shape=(
            jax.ShapeDtypeStruct((_B, 1, _U), jnp.float32),
            jax.ShapeDtypeStruct((_B, 1, _U), jnp.float32),
        ),
    )(counts2, x3, kb3, vc, ipw_t, ipb2, opw_t, opb2, ln1g2, ln1b2,
      w1_t, b12, w2_t, b22, ln2g2, ln2b2)


def _final_body(vs_ref, pr_ref, cnt_ref, d1_ref, b1_ref, d2_ref, b2_ref,
                out_ref):
    cnt = cnt_ref[...].astype(jnp.float32)
    mx = jnp.max(cnt)
    pooled = (vs_ref[...] + (mx - cnt) * pr_ref[...]) / mx
    hid = jnp.maximum(
        jnp.dot(pooled, d1_ref[...], preferred_element_type=jnp.float32)
        + b1_ref[...], 0.0)
    logit = jnp.dot(hid, d2_ref[...],
                    preferred_element_type=jnp.float32) + b2_ref[...]
    out_ref[...] = jax.nn.sigmoid(logit)


def _final(vs, pr, counts, d1_t, d1b2, d2_t, d2b2):
    return pl.pallas_call(
        _final_body,
        out_shape=jax.ShapeDtypeStruct((_B, 1), jnp.float32),
    )(vs, pr, counts, d1_t, d1b2, d2_t, d2b2)


def kernel(atom_features, bond_features, kernel, bias_p, w_ih, w_hh, b_ih,
           b_hh, ipw, ipb, opw, opb, ln1_g, ln1_b, w1, b1, w2, b2, ln2_g,
           ln2_b, d1_w, d1_b, d2_w, d2_b, pair_indices, molecule_indicator):
    src = pair_indices[:, 1]
    dst2d = pair_indices[:, 0].reshape(_NW * _NCH, _CHUNK)
    bfa = jnp.concatenate(
        [bond_features, jnp.ones((_E, 1), jnp.float32)], axis=1)
    k3 = jnp.concatenate([kernel, bias_p[None, :]], axis=0)
    kt2 = k3.reshape(17, _U, _U).transpose(2, 0, 1).reshape(_U, 17 * _U)
    zeros_nu = jnp.zeros((_N, _U), jnp.float32)

    gidx, validf, counts, keybias = _plan(molecule_indicator.reshape(1, _N))

    h = atom_features
    for _ in range(4):
        neigh = _sc_gather(h, src)
        msg = _msg(neigh, bfa, kt2)
        partials = _sc_scatter(msg, dst2d, zeros_nu)
        h = _gru(partials, h, w_ih.T, w_hh.T, b_ih[None, :], b_hh[None, :])

    xg = _sc_gather(h, gidx.reshape(_E))
    vs, pr = _attn(
        counts.reshape(_B, 1, 1), xg.reshape(_B, _L, _U),
        keybias.reshape(_B, 1, _L),
        validf.reshape(_B, _L, 1),
        ipw.T, ipb[None, :], opw.T, opb[None, :],
        ln1_g[None, :], ln1_b[None, :], w1.T, b1[None, :],
        w2.T, b2[None, :], ln2_g[None, :], ln2_b[None, :])
    return _final(vs.reshape(_B, _U), pr.reshape(_B, _U), counts, d1_w.T,
                  d1_b[None, :], d2_w.T, d2_b[None, :])


# speedup vs baseline: 2.2543x; 1.0268x over previous
import functools

import jax
import jax.numpy as jnp
from jax import lax
from jax.experimental import pallas as pl
from jax.experimental.pallas import tpu as pltpu
from jax.experimental.pallas import tpu_sc as plsc

_E = 16384
_N = 4096
_U = 64
_B = 32
_L = 512
_NC, _NS = 2, 16
_NW = _NC * _NS
_EPW = _E // _NW
_CHUNK = 128
_NCH = _EPW // _CHUNK
_RPS = _N // _NS

_SC_MESH = plsc.VectorSubcoreMesh(core_axis_name="c", subcore_axis_name="s")
_SC_PARAMS = pltpu.CompilerParams(use_tc_tiling_on_sc=False)


def _gather_body(table_hbm, idx_hbm, out_hbm, idx_v, rows_v, sem):
    wid = lax.axis_index("s") * _NC + lax.axis_index("c")
    base = wid * _EPW
    pltpu.sync_copy(idx_hbm.at[pl.ds(base, _EPW)], idx_v)
    descs = [
        pltpu.async_copy(
            table_hbm.at[idx_v.at[pl.ds(j * _CHUNK, _CHUNK)]],
            rows_v.at[pl.ds(j * _CHUNK, _CHUNK)], sem)
        for j in range(_NCH)
    ]
    for d in descs:
        d.wait()
    pltpu.sync_copy(rows_v, out_hbm.at[pl.ds(base, _EPW)])


def _sc_gather(table, idx):
    return pl.kernel(
        _gather_body,
        out_type=jax.ShapeDtypeStruct((_E, _U), jnp.float32),
        mesh=_SC_MESH,
        scratch_types=[
            pltpu.VMEM((_EPW,), jnp.int32),
            pltpu.VMEM((_EPW, _U), jnp.float32),
            pltpu.SemaphoreType.DMA,
        ],
        compiler_params=_SC_PARAMS,
    )(table, idx)


def _scatter_body(msg_hbm, dst_hbm, zero_hbm, out_hbm, idx_v, msg_v, agg_sh):
    cid = lax.axis_index("c")
    sid = lax.axis_index("s")
    wid = sid * _NC + cid
    rows0 = sid * _RPS
    pltpu.sync_copy(zero_hbm.at[pl.ds(rows0, _RPS)],
                    agg_sh.at[pl.ds(rows0, _RPS)])
    plsc.subcore_barrier()
    base = wid * _EPW
    pltpu.sync_copy(msg_hbm.at[pl.ds(base, _EPW)], msg_v)
    pltpu.sync_copy(dst_hbm.at[pl.ds(wid * _NCH, _NCH)], idx_v)
    for j in range(_NCH):
        pltpu.sync_copy(msg_v.at[pl.ds(j * _CHUNK, _CHUNK)],
                        agg_sh.at[idx_v.at[j]], add=True)
    plsc.subcore_barrier()
    pltpu.sync_copy(agg_sh.at[pl.ds(rows0, _RPS)],
                    out_hbm.at[pl.ds(cid * _N + rows0, _RPS)])


def _sc_scatter(msg, dst2d, zeros_nu):
    return pl.kernel(
        _scatter_body,
        out_type=jax.ShapeDtypeStruct((_NC * _N, _U), jnp.float32),
        mesh=_SC_MESH,
        scratch_types=[
            pltpu.VMEM((_NCH, _CHUNK), jnp.int32),
            pltpu.VMEM((_EPW, _U), jnp.float32),
            pltpu.VMEM_SHARED((_N, _U), jnp.float32),
        ],
        compiler_params=_SC_PARAMS,
    )(msg, dst2d, zeros_nu)


def _plan_body(mol_ref, gidx_ref, valid_ref, counts_ref, kb_ref):
    mol = mol_ref[...]
    mids = lax.broadcasted_iota(jnp.int32, (_B, _N), 0)
    counts = jnp.sum((mol == mids).astype(jnp.int32), axis=1, keepdims=True)
    starts = jnp.sum((mids > mol).astype(jnp.int32), axis=1, keepdims=True)
    p = lax.broadcasted_iota(jnp.int32, (_B, _L), 1)
    mrow = lax.broadcasted_iota(jnp.int32, (_B, _L), 0)
    lim = jnp.minimum(counts, _L)
    valid = (p < counts).astype(jnp.float32)
    valid_ref[...] = valid
    gidx_ref[...] = jnp.where(p < lim, starts + p,
                              jnp.bitwise_and(mrow * _L + p, _N - 1))
    counts_ref[...] = counts
    kb_ref[...] = (valid - 1.0) * jnp.float32(1e9)


def _plan(mol2d):
    return pl.pallas_call(
        _plan_body,
        out_shape=(
            jax.ShapeDtypeStruct((_B, _L), jnp.int32),
            jax.ShapeDtypeStruct((_B, _L), jnp.float32),
            jax.ShapeDtypeStruct((_B, 1), jnp.int32),
            jax.ShapeDtypeStruct((_B, _L), jnp.float32),
        ),
    )(mol2d)


def _msg_body(neigh_ref, bfa_ref, kt_ref, out_ref):
    p = jnp.dot(neigh_ref[...], kt_ref[...],
                preferred_element_type=jnp.float32)
    acc = bfa_ref[:, 0:1] * p[:, 0:_U]
    for b in range(1, 17):
        acc = acc + bfa_ref[:, b:b + 1] * p[:, b * _U:(b + 1) * _U]
    out_ref[...] = acc


def _msg(neigh, bfa, kt2):
    blk = 2048
    return pl.pallas_call(
        _msg_body,
        grid=(_E // blk,),
        in_specs=[
            pl.BlockSpec((blk, _U), lambda i: (i, 0)),
            pl.BlockSpec((blk, 17), lambda i: (i, 0)),
            pl.BlockSpec((_U, 17 * _U), lambda i: (0, 0)),
        ],
        out_specs=pl.BlockSpec((blk, _U), lambda i: (i, 0)),
        out_shape=jax.ShapeDtypeStruct((_E, _U), jnp.float32),
    )(neigh, bfa, kt2)


def _gru_body(p0_ref, p1_ref, h_ref, wih_ref, whh_ref, bih_ref, bhh_ref,
              out_ref):
    agg = p0_ref[...] + p1_ref[...]
    gi = jnp.dot(agg, wih_ref[...],
                 preferred_element_type=jnp.float32) + bih_ref[...]
    gh = jnp.dot(h_ref[...], whh_ref[...],
                 preferred_element_type=jnp.float32) + bhh_ref[...]
    r = jax.nn.sigmoid(gi[:, 0:_U] + gh[:, 0:_U])
    z = jax.nn.sigmoid(gi[:, _U:2 * _U] + gh[:, _U:2 * _U])
    n = jnp.tanh(gi[:, 2 * _U:3 * _U] + r * gh[:, 2 * _U:3 * _U])
    out_ref[...] = (1.0 - z) * n + z * h_ref[...]


def _gru(partials, h, wih_t, whh_t, bih2, bhh2):
    blk = 1024
    nb = _N // blk
    return pl.pallas_call(
        _gru_body,
        grid=(nb,),
        in_specs=[
            pl.BlockSpec((blk, _U), lambda i: (i, 0)),
            pl.BlockSpec((blk, _U), lambda i, _nb=nb: (i + _nb, 0)),
            pl.BlockSpec((blk, _U), lambda i: (i, 0)),
            pl.BlockSpec((_U, 3 * _U), lambda i: (0, 0)),
            pl.BlockSpec((_U, 3 * _U), lambda i: (0, 0)),
            pl.BlockSpec((1, 3 * _U), lambda i: (0, 0)),
            pl.BlockSpec((1, 3 * _U), lambda i: (0, 0)),
        ],
        out_specs=pl.BlockSpec((blk, _U), lambda i: (i, 0)),
        out_shape=jax.ShapeDtypeStruct((_N, _U), jnp.float32),
    )(partials, partials, h, wih_t, whh_t, bih2, bhh2)


_RT = 256
_PADR = 8


def _attn_body(cnt_ref, x_ref, kb_ref, vc_ref, ipw_ref, ipb_ref, opw_ref,
               opb_ref, ln1g_ref, ln1b_ref, w1_ref, b1_ref, w2_ref, b2_ref,
               ln2g_ref, ln2b_ref, vs_ref, pr_ref):
    hd = _U // 8
    count = cnt_ref[0, 0, 0]
    x = x_ref[0] * vc_ref[0]
    kb = kb_ref[0]
    qkv = jnp.dot(x, ipw_ref[...],
                  preferred_element_type=jnp.float32) + ipb_ref[...]
    scale = 1.0 / jnp.sqrt(jnp.float32(hd))
    q = qkv[:, 0:_U] * scale
    k = qkv[:, _U:2 * _U]
    v = qkv[:, 2 * _U:3 * _U]

    def tile(r0, rows):
        qt = q[r0:r0 + rows, :]
        ctx_parts = []
        for h in range(8):
            qh = qt[:, h * hd:(h + 1) * hd]
            kh = k[:, h * hd:(h + 1) * hd]
            vh = v[:, h * hd:(h + 1) * hd]
            s = lax.dot_general(qh, kh, (((1,), (1,)), ((), ())),
                                preferred_element_type=jnp.float32) + kb
            m = jnp.max(s, axis=1, keepdims=True)
            e = jnp.exp(s - m)
            a = e / jnp.sum(e, axis=1, keepdims=True)
            ctx_parts.append(
                jnp.dot(a, vh, preferred_element_type=jnp.float32))
        ctx = jnp.concatenate(ctx_parts, axis=1)
        attn_out = jnp.dot(ctx, opw_ref[...],
                           preferred_element_type=jnp.float32) + opb_ref[...]
        y = x[r0:r0 + rows, :] + attn_out
        mu = jnp.mean(y, axis=1, keepdims=True)
        var = jnp.mean((y - mu) ** 2, axis=1, keepdims=True)
        pin = ((y - mu) / jnp.sqrt(var + 1e-5) * ln1g_ref[...]
               + ln1b_ref[...])
        hid = jnp.maximum(
            jnp.dot(pin, w1_ref[...], preferred_element_type=jnp.float32)
            + b1_ref[...], 0.0)
        mlp = jnp.dot(hid, w2_ref[...],
                      preferred_element_type=jnp.float32) + b2_ref[...]
        y2 = pin + mlp
        mu2 = jnp.mean(y2, axis=1, keepdims=True)
        var2 = jnp.mean((y2 - mu2) ** 2, axis=1, keepdims=True)
        return ((y2 - mu2) / jnp.sqrt(var2 + 1e-5) * ln2g_ref[...]
                + ln2b_ref[...])

    vs_ref[0] = jnp.zeros((1, _U), jnp.float32)
    for r in range(_L // _RT):
        @pl.when(count > r * _RT)
        def _():
            pout = tile(r * _RT, _RT)
            vm = vc_ref[0][r * _RT:(r + 1) * _RT, :]
            vs_ref[0] = vs_ref[0] + jnp.sum(pout * vm, axis=0, keepdims=True)
    pout_pad = tile(_L - _PADR, _PADR)
    pr_ref[0] = pout_pad[_PADR - 1:_PADR, :]


def _attn(counts2, x3, kb3, vc, ipw_t, ipb2, opw_t, opb2, ln1g2, ln1b2,
          w1_t, b12, w2_t, b22, ln2g2, ln2b2):
    full = lambda a, b: pl.BlockSpec((a, b), lambda i: (0, 0))
    return pl.pallas_call(
        _attn_body,
        grid=(_B,),
        in_specs=[
            pl.BlockSpec((1, 1, 1), lambda i: (i, 0, 0),
                         memory_space=pltpu.SMEM),
            pl.BlockSpec((1, _L, _U), lambda i: (i, 0, 0)),
            pl.BlockSpec((1, 1, _L), lambda i: (i, 0, 0)),
            pl.BlockSpec((1, _L, 1), lambda i: (i, 0, 0)),
            full(_U, 3 * _U), full(1, 3 * _U),
            full(_U, _U), full(1, _U),
            full(1, _U), full(1, _U),
            full(_U, _U), full(1, _U),
            full(_U, _U), full(1, _U),
            full(1, _U), full(1, _U),
        ],
        out_specs=(
            pl.BlockSpec((1, 1, _U), lambda i: (i, 0, 0)),
            pl.BlockSpec((1, 1, _U), lambda i: (i, 0, 0)),
        ),
        out_shape=(
            jax.ShapeDtypeStruct((_B, 1, _U), jnp.float32),
            jax.ShapeDtypeStruct((_B, 1, _U), jnp.float32),
        ),
    )(counts2, x3, kb3, vc, ipw_t, ipb2, opw_t, opb2, ln1g2, ln1b2,
      w1_t, b12, w2_t, b22, ln2g2, ln2b2)


def _final_body(vs_ref, pr_ref, cnt_ref, d1_ref, b1_ref, d2_ref, b2_ref,
                out_ref):
    cnt = cnt_ref[...].astype(jnp.float32)
    mx = jnp.max(cnt)
    pooled = (vs_ref[...] + (mx - cnt) * pr_ref[...]) / mx
    hid = jnp.maximum(
        jnp.dot(pooled, d1_ref[...], preferred_element_type=jnp.float32)
        + b1_ref[...], 0.0)
    logit = jnp.dot(hid, d2_ref[...],
                    preferred_element_type=jnp.float32) + b2_ref[...]
    out_ref[...] = jax.nn.sigmoid(logit)


def _final(vs, pr, counts, d1_t, d1b2, d2_t, d2b2):
    return pl.pallas_call(
        _final_body,
        out_shape=jax.ShapeDtypeStruct((_B, 1), jnp.float32),
    )(vs, pr, counts, d1_t, d1b2, d2_t, d2b2)


def kernel(atom_features, bond_features, kernel, bias_p, w_ih, w_hh, b_ih,
           b_hh, ipw, ipb, opw, opb, ln1_g, ln1_b, w1, b1, w2, b2, ln2_g,
           ln2_b, d1_w, d1_b, d2_w, d2_b, pair_indices, molecule_indicator):
    src = pair_indices[:, 1]
    dst2d = pair_indices[:, 0].reshape(_NW * _NCH, _CHUNK)
    bfa = jnp.concatenate(
        [bond_features, jnp.ones((_E, 1), jnp.float32)], axis=1)
    k3 = jnp.concatenate([kernel, bias_p[None, :]], axis=0)
    kt2 = k3.reshape(17, _U, _U).transpose(2, 0, 1).reshape(_U, 17 * _U)
    zeros_nu = jnp.zeros((_N, _U), jnp.float32)

    gidx, validf, counts, keybias = _plan(molecule_indicator.reshape(1, _N))

    h = atom_features
    for _ in range(4):
        neigh = _sc_gather(h, src)
        msg = _msg(neigh, bfa, kt2)
        partials = _sc_scatter(msg, dst2d, zeros_nu)
        h = _gru(partials, h, w_ih.T, w_hh.T, b_ih[None, :], b_hh[None, :])

    xg = _sc_gather(h, gidx.reshape(_E))
    vs, pr = _attn(
        counts.reshape(_B, 1, 1), xg.reshape(_B, _L, _U),
        keybias.reshape(_B, 1, _L),
        validf.reshape(_B, _L, 1),
        ipw.T, ipb[None, :], opw.T, opb[None, :],
        ln1_g[None, :], ln1_b[None, :], w1.T, b1[None, :],
        w2.T, b2[None, :], ln2_g[None, :], ln2_b[None, :])
    return _final(vs.reshape(_B, _U), pr.reshape(_B, _U), counts, d1_w.T,
                  d1_b[None, :], d2_w.T, d2_b[None, :])


# speedup vs baseline: 2.4577x; 1.0902x over previous
import functools

import jax
import jax.numpy as jnp
from jax import lax
from jax.experimental import pallas as pl
from jax.experimental.pallas import tpu as pltpu
from jax.experimental.pallas import tpu_sc as plsc

_E = 16384
_N = 4096
_U = 64
_B = 32
_L = 512
_NC, _NS = 2, 16
_NW = _NC * _NS
_EPW = _E // _NW
_CHUNK = 128
_NCH = _EPW // _CHUNK
_RPS = _N // _NS

_UP = 128
_SC_MESH = plsc.VectorSubcoreMesh(core_axis_name="c", subcore_axis_name="s")


def _gather_body(table_hbm, idx_hbm, out_hbm, idx_v, rows_v, sem):
    wid = lax.axis_index("s") * _NC + lax.axis_index("c")
    base = wid * _EPW
    pltpu.sync_copy(idx_hbm.at[pl.ds(base, _EPW)], idx_v)
    descs = [
        pltpu.async_copy(
            table_hbm.at[idx_v.at[pl.ds(j * _CHUNK, _CHUNK)]],
            rows_v.at[pl.ds(j * _CHUNK, _CHUNK)], sem)
        for j in range(_NCH)
    ]
    for d in descs:
        d.wait()
    pltpu.sync_copy(rows_v, out_hbm.at[pl.ds(base, _EPW)])


def _sc_gather(table, idx):
    return pl.kernel(
        _gather_body,
        out_type=jax.ShapeDtypeStruct((_E, _UP), jnp.float32),
        mesh=_SC_MESH,
        scratch_types=[
            pltpu.VMEM((_EPW,), jnp.int32),
            pltpu.VMEM((_EPW, _UP), jnp.float32),
            pltpu.SemaphoreType.DMA,
        ],
    )(table, idx)


def _scatter_body(msg_hbm, dst_hbm, zero_hbm, out_hbm, idx_v, msg_v, agg_sh):
    cid = lax.axis_index("c")
    sid = lax.axis_index("s")
    wid = sid * _NC + cid
    rows0 = sid * _RPS
    pltpu.sync_copy(zero_hbm.at[pl.ds(rows0, _RPS)],
                    agg_sh.at[pl.ds(rows0, _RPS)])
    plsc.subcore_barrier()
    base = wid * _EPW
    pltpu.sync_copy(msg_hbm.at[pl.ds(base, _EPW)], msg_v)
    pltpu.sync_copy(dst_hbm.at[pl.ds(wid * _NCH, _NCH)], idx_v)
    for j in range(_NCH):
        pltpu.sync_copy(msg_v.at[pl.ds(j * _CHUNK, _CHUNK)],
                        agg_sh.at[idx_v.at[j]], add=True)
    plsc.subcore_barrier()
    pltpu.sync_copy(agg_sh.at[pl.ds(rows0, _RPS)],
                    out_hbm.at[pl.ds(cid * _N + rows0, _RPS)])


def _sc_scatter(msg, dst2d, zeros_nu):
    return pl.kernel(
        _scatter_body,
        out_type=jax.ShapeDtypeStruct((_NC * _N, _UP), jnp.float32),
        mesh=_SC_MESH,
        scratch_types=[
            pltpu.VMEM((_NCH, _CHUNK), jnp.int32),
            pltpu.VMEM((_EPW, _UP), jnp.float32),
            pltpu.VMEM_SHARED((_N, _UP), jnp.float32),
        ],
    )(msg, dst2d, zeros_nu)


def _plan_body(mol_ref, gidx_ref, valid_ref, counts_ref, kb_ref):
    mol = mol_ref[...]
    mids = lax.broadcasted_iota(jnp.int32, (_B, _N), 0)
    counts = jnp.sum((mol == mids).astype(jnp.int32), axis=1, keepdims=True)
    starts = jnp.sum((mids > mol).astype(jnp.int32), axis=1, keepdims=True)
    p = lax.broadcasted_iota(jnp.int32, (_B, _L), 1)
    mrow = lax.broadcasted_iota(jnp.int32, (_B, _L), 0)
    lim = jnp.minimum(counts, _L)
    valid = (p < counts).astype(jnp.float32)
    valid_ref[...] = valid
    gidx_ref[...] = jnp.where(p < lim, starts + p,
                              jnp.bitwise_and(mrow * _L + p, _N - 1))
    counts_ref[...] = counts
    kb_ref[...] = (valid - 1.0) * jnp.float32(1e9)


def _plan(mol2d):
    return pl.pallas_call(
        _plan_body,
        out_shape=(
            jax.ShapeDtypeStruct((_B, _L), jnp.int32),
            jax.ShapeDtypeStruct((_B, _L), jnp.float32),
            jax.ShapeDtypeStruct((_B, 1), jnp.int32),
            jax.ShapeDtypeStruct((_B, _L), jnp.float32),
        ),
    )(mol2d)


def _msg_body(neigh_ref, bfa_ref, kt_ref, out_ref):
    p = jnp.dot(neigh_ref[:, 0:_U], kt_ref[...],
                preferred_element_type=jnp.float32)
    acc = bfa_ref[:, 0:1] * p[:, 0:_U]
    for b in range(1, 17):
        acc = acc + bfa_ref[:, b:b + 1] * p[:, b * _U:(b + 1) * _U]
    out_ref[...] = jnp.concatenate([acc, jnp.zeros_like(acc)], axis=1)


def _msg(neigh, bfa, kt2):
    blk = 2048
    return pl.pallas_call(
        _msg_body,
        grid=(_E // blk,),
        in_specs=[
            pl.BlockSpec((blk, _UP), lambda i: (i, 0)),
            pl.BlockSpec((blk, 17), lambda i: (i, 0)),
            pl.BlockSpec((_U, 17 * _U), lambda i: (0, 0)),
        ],
        out_specs=pl.BlockSpec((blk, _UP), lambda i: (i, 0)),
        out_shape=jax.ShapeDtypeStruct((_E, _UP), jnp.float32),
    )(neigh, bfa, kt2)


def _gru_body(p0_ref, p1_ref, h_ref, wih_ref, whh_ref, bih_ref, bhh_ref,
              out_ref):
    agg = p0_ref[:, 0:_U] + p1_ref[:, 0:_U]
    hh = h_ref[:, 0:_U]
    gi = jnp.dot(agg, wih_ref[...],
                 preferred_element_type=jnp.float32) + bih_ref[...]
    gh = jnp.dot(hh, whh_ref[...],
                 preferred_element_type=jnp.float32) + bhh_ref[...]
    r = jax.nn.sigmoid(gi[:, 0:_U] + gh[:, 0:_U])
    z = jax.nn.sigmoid(gi[:, _U:2 * _U] + gh[:, _U:2 * _U])
    n = jnp.tanh(gi[:, 2 * _U:3 * _U] + r * gh[:, 2 * _U:3 * _U])
    hnew = (1.0 - z) * n + z * hh
    out_ref[...] = jnp.concatenate([hnew, jnp.zeros_like(hnew)], axis=1)


def _gru(partials, h, wih_t, whh_t, bih2, bhh2):
    blk = 1024
    nb = _N // blk
    return pl.pallas_call(
        _gru_body,
        grid=(nb,),
        in_specs=[
            pl.BlockSpec((blk, _UP), lambda i: (i, 0)),
            pl.BlockSpec((blk, _UP), lambda i, _nb=nb: (i + _nb, 0)),
            pl.BlockSpec((blk, _UP), lambda i: (i, 0)),
            pl.BlockSpec((_U, 3 * _U), lambda i: (0, 0)),
            pl.BlockSpec((_U, 3 * _U), lambda i: (0, 0)),
            pl.BlockSpec((1, 3 * _U), lambda i: (0, 0)),
            pl.BlockSpec((1, 3 * _U), lambda i: (0, 0)),
        ],
        out_specs=pl.BlockSpec((blk, _UP), lambda i: (i, 0)),
        out_shape=jax.ShapeDtypeStruct((_N, _UP), jnp.float32),
    )(partials, partials, h, wih_t, whh_t, bih2, bhh2)


_RT = 256
_PADR = 8


def _attn_body(cnt_ref, x_ref, kb_ref, vc_ref, ipw_ref, ipb_ref, opw_ref,
               opb_ref, ln1g_ref, ln1b_ref, w1_ref, b1_ref, w2_ref, b2_ref,
               ln2g_ref, ln2b_ref, vs_ref, pr_ref):
    hd = _U // 8
    count = cnt_ref[0, 0, 0]
    x = x_ref[0][:, 0:_U] * vc_ref[0]
    kb = kb_ref[0]
    qkv = jnp.dot(x, ipw_ref[...],
                  preferred_element_type=jnp.float32) + ipb_ref[...]
    scale = 1.0 / jnp.sqrt(jnp.float32(hd))
    q = qkv[:, 0:_U] * scale
    k = qkv[:, _U:2 * _U]
    v = qkv[:, 2 * _U:3 * _U]

    def tile(r0, rows):
        qt = q[r0:r0 + rows, :]
        ctx_parts = []
        for h in range(8):
            qh = qt[:, h * hd:(h + 1) * hd]
            kh = k[:, h * hd:(h + 1) * hd]
            vh = v[:, h * hd:(h + 1) * hd]
            s = lax.dot_general(qh, kh, (((1,), (1,)), ((), ())),
                                preferred_element_type=jnp.float32) + kb
            m = jnp.max(s, axis=1, keepdims=True)
            e = jnp.exp(s - m)
            a = e / jnp.sum(e, axis=1, keepdims=True)
            ctx_parts.append(
                jnp.dot(a, vh, preferred_element_type=jnp.float32))
        ctx = jnp.concatenate(ctx_parts, axis=1)
        attn_out = jnp.dot(ctx, opw_ref[...],
                           preferred_element_type=jnp.float32) + opb_ref[...]
        y = x[r0:r0 + rows, :] + attn_out
        mu = jnp.mean(y, axis=1, keepdims=True)
        var = jnp.mean((y - mu) ** 2, axis=1, keepdims=True)
        pin = ((y - mu) / jnp.sqrt(var + 1e-5) * ln1g_ref[...]
               + ln1b_ref[...])
        hid = jnp.maximum(
            jnp.dot(pin, w1_ref[...], preferred_element_type=jnp.float32)
            + b1_ref[...], 0.0)
        mlp = jnp.dot(hid, w2_ref[...],
                      preferred_element_type=jnp.float32) + b2_ref[...]
        y2 = pin + mlp
        mu2 = jnp.mean(y2, axis=1, keepdims=True)
        var2 = jnp.mean((y2 - mu2) ** 2, axis=1, keepdims=True)
        return ((y2 - mu2) / jnp.sqrt(var2 + 1e-5) * ln2g_ref[...]
                + ln2b_ref[...])

    vs_ref[0] = jnp.zeros((1, _U), jnp.float32)
    for r in range(_L // _RT):
        @pl.when(count > r * _RT)
        def _():
            pout = tile(r * _RT, _RT)
            vm = vc_ref[0][r * _RT:(r + 1) * _RT, :]
            vs_ref[0] = vs_ref[0] + jnp.sum(pout * vm, axis=0, keepdims=True)
    pout_pad = tile(_L - _PADR, _PADR)
    pr_ref[0] = pout_pad[_PADR - 1:_PADR, :]


def _attn(counts2, x3, kb3, vc, ipw_t, ipb2, opw_t, opb2, ln1g2, ln1b2,
          w1_t, b12, w2_t, b22, ln2g2, ln2b2):
    full = lambda a, b: pl.BlockSpec((a, b), lambda i: (0, 0))
    return pl.pallas_call(
        _attn_body,
        grid=(_B,),
        in_specs=[
            pl.BlockSpec((1, 1, 1), lambda i: (i, 0, 0),
                         memory_space=pltpu.SMEM),
            pl.BlockSpec((1, _L, _UP), lambda i: (i, 0, 0)),
            pl.BlockSpec((1, 1, _L), lambda i: (i, 0, 0)),
            pl.BlockSpec((1, _L, 1), lambda i: (i, 0, 0)),
            full(_U, 3 * _U), full(1, 3 * _U),
            full(_U, _U), full(1, _U),
            full(1, _U), full(1, _U),
            full(_U, _U), full(1, _U),
            full(_U, _U), full(1, _U),
            full(1, _U), full(1, _U),
        ],
        out_specs=(
            pl.BlockSpec((1, 1, _U), lambda i: (i, 0, 0)),
            pl.BlockSpec((1, 1, _U), lambda i: (i, 0, 0)),
        ),
        out_shape=(
            jax.ShapeDtypeStruct((_B, 1, _U), jnp.float32),
            jax.ShapeDtypeStruct((_B, 1, _U), jnp.float32),
        ),
    )(counts2, x3, kb3, vc, ipw_t, ipb2, opw_t, opb2, ln1g2, ln1b2,
      w1_t, b12, w2_t, b22, ln2g2, ln2b2)


def _final_body(vs_ref, pr_ref, cnt_ref, d1_ref, b1_ref, d2_ref, b2_ref,
                out_ref):
    cnt = cnt_ref[...].astype(jnp.float32)
    mx = jnp.max(cnt)
    pooled = (vs_ref[...] + (mx - cnt) * pr_ref[...]) / mx
    hid = jnp.maximum(
        jnp.dot(pooled, d1_ref[...], preferred_element_type=jnp.float32)
        + b1_ref[...], 0.0)
    logit = jnp.dot(hid, d2_ref[...],
                    preferred_element_type=jnp.float32) + b2_ref[...]
    out_ref[...] = jax.nn.sigmoid(logit)


def _final(vs, pr, counts, d1_t, d1b2, d2_t, d2b2):
    return pl.pallas_call(
        _final_body,
        out_shape=jax.ShapeDtypeStruct((_B, 1), jnp.float32),
    )(vs, pr, counts, d1_t, d1b2, d2_t, d2b2)


def kernel(atom_features, bond_features, kernel, bias_p, w_ih, w_hh, b_ih,
           b_hh, ipw, ipb, opw, opb, ln1_g, ln1_b, w1, b1, w2, b2, ln2_g,
           ln2_b, d1_w, d1_b, d2_w, d2_b, pair_indices, molecule_indicator):
    src = pair_indices[:, 1]
    dst2d = pair_indices[:, 0].reshape(_NW * _NCH, _CHUNK)
    bfa = jnp.concatenate(
        [bond_features, jnp.ones((_E, 1), jnp.float32)], axis=1)
    k3 = jnp.concatenate([kernel, bias_p[None, :]], axis=0)
    kt2 = k3.reshape(17, _U, _U).transpose(2, 0, 1).reshape(_U, 17 * _U)
    zeros_nu = jnp.zeros((_N, _UP), jnp.float32)

    gidx, validf, counts, keybias = _plan(molecule_indicator.reshape(1, _N))

    h = jnp.pad(atom_features, ((0, 0), (0, _UP - _U)))
    for _ in range(4):
        neigh = _sc_gather(h, src)
        msg = _msg(neigh, bfa, kt2)
        partials = _sc_scatter(msg, dst2d, zeros_nu)
        h = _gru(partials, h, w_ih.T, w_hh.T, b_ih[None, :], b_hh[None, :])

    xg = _sc_gather(h, gidx.reshape(_E))
    vs, pr = _attn(
        counts.reshape(_B, 1, 1), xg.reshape(_B, _L, _UP),
        keybias.reshape(_B, 1, _L),
        validf.reshape(_B, _L, 1),
        ipw.T, ipb[None, :], opw.T, opb[None, :],
        ln1_g[None, :], ln1_b[None, :], w1.T, b1[None, :],
        w2.T, b2[None, :], ln2_g[None, :], ln2_b[None, :])
    return _final(vs.reshape(_B, _U), pr.reshape(_B, _U), counts, d1_w.T,
                  d1_b[None, :], d2_w.T, d2_b[None, :])


# speedup vs baseline: 2.4619x; 1.0017x over previous
import functools

import jax
import jax.numpy as jnp
from jax import lax
from jax.experimental import pallas as pl
from jax.experimental.pallas import tpu as pltpu
from jax.experimental.pallas import tpu_sc as plsc

_E = 16384
_N = 4096
_U = 64
_B = 32
_L = 512
_NC, _NS = 2, 16
_NW = _NC * _NS
_EPW = _E // _NW
_CHUNK = 128
_NCH = _EPW // _CHUNK
_RPS = _N // _NS

_UP = 128
_SC_MESH = plsc.VectorSubcoreMesh(core_axis_name="c", subcore_axis_name="s")


def _gather_body(table_hbm, idx_hbm, out_hbm, idx_v, rows_v, sem):
    wid = lax.axis_index("s") * _NC + lax.axis_index("c")
    base = wid * _EPW
    pltpu.sync_copy(idx_hbm.at[pl.ds(base, _EPW)], idx_v)
    descs = [
        pltpu.async_copy(
            table_hbm.at[idx_v.at[pl.ds(j * _CHUNK, _CHUNK)]],
            rows_v.at[pl.ds(j * _CHUNK, _CHUNK)], sem)
        for j in range(_NCH)
    ]
    for d in descs:
        d.wait()
    pltpu.sync_copy(rows_v, out_hbm.at[pl.ds(base, _EPW)])


def _sc_gather(table, idx):
    return pl.kernel(
        _gather_body,
        out_type=jax.ShapeDtypeStruct((_E, _UP), jnp.float32),
        mesh=_SC_MESH,
        scratch_types=[
            pltpu.VMEM((_EPW,), jnp.int32),
            pltpu.VMEM((_EPW, _UP), jnp.float32),
            pltpu.SemaphoreType.DMA,
        ],
    )(table, idx)


def _scatter_body(msg_hbm, dst_hbm, zero_hbm, out_hbm, idx_v, msg_v, agg_sh):
    cid = lax.axis_index("c")
    sid = lax.axis_index("s")
    wid = sid * _NC + cid
    rows0 = sid * _RPS
    pltpu.sync_copy(zero_hbm.at[pl.ds(rows0, _RPS)],
                    agg_sh.at[pl.ds(rows0, _RPS)])
    plsc.subcore_barrier()
    base = wid * _EPW
    pltpu.sync_copy(msg_hbm.at[pl.ds(base, _EPW)], msg_v)
    pltpu.sync_copy(dst_hbm.at[pl.ds(wid * _NCH, _NCH)], idx_v)
    for j in range(_NCH):
        pltpu.sync_copy(msg_v.at[pl.ds(j * _CHUNK, _CHUNK)],
                        agg_sh.at[idx_v.at[j]], add=True)
    plsc.subcore_barrier()
    pltpu.sync_copy(agg_sh.at[pl.ds(rows0, _RPS)],
                    out_hbm.at[pl.ds(cid * _N + rows0, _RPS)])


def _sc_scatter(msg, dst2d, zeros_nu):
    return pl.kernel(
        _scatter_body,
        out_type=jax.ShapeDtypeStruct((_NC * _N, _UP), jnp.float32),
        mesh=_SC_MESH,
        scratch_types=[
            pltpu.VMEM((_NCH, _CHUNK), jnp.int32),
            pltpu.VMEM((_EPW, _UP), jnp.float32),
            pltpu.VMEM_SHARED((_N, _UP), jnp.float32),
        ],
    )(msg, dst2d, zeros_nu)


def _plan_body(mol_ref, gidx_ref, valid_ref, counts_ref, kb_ref):
    mol = mol_ref[...]
    mids = lax.broadcasted_iota(jnp.int32, (_B, _N), 0)
    counts = jnp.sum((mol == mids).astype(jnp.int32), axis=1, keepdims=True)
    starts = jnp.sum((mids > mol).astype(jnp.int32), axis=1, keepdims=True)
    p = lax.broadcasted_iota(jnp.int32, (_B, _L), 1)
    mrow = lax.broadcasted_iota(jnp.int32, (_B, _L), 0)
    lim = jnp.minimum(counts, _L)
    valid = (p < counts).astype(jnp.float32)
    valid_ref[...] = valid
    gidx_ref[...] = jnp.where(p < lim, starts + p,
                              jnp.bitwise_and(mrow * _L + p, _N - 1))
    counts_ref[...] = counts
    kb_ref[...] = (valid - 1.0) * jnp.float32(1e9)


def _plan(mol2d):
    return pl.pallas_call(
        _plan_body,
        out_shape=(
            jax.ShapeDtypeStruct((_B, _L), jnp.int32),
            jax.ShapeDtypeStruct((_B, _L), jnp.float32),
            jax.ShapeDtypeStruct((_B, 1), jnp.int32),
            jax.ShapeDtypeStruct((_B, _L), jnp.float32),
        ),
    )(mol2d)


def _msg_body(neigh_ref, bfa_ref, kt_ref, out_ref):
    nb = neigh_ref[:, 0:_U].astype(jnp.bfloat16)
    p = jnp.dot(nb, kt_ref[...],
                preferred_element_type=jnp.float32)
    acc = bfa_ref[:, 0:1] * p[:, 0:_U]
    for b in range(1, 17):
        acc = acc + bfa_ref[:, b:b + 1] * p[:, b * _U:(b + 1) * _U]
    out_ref[...] = jnp.concatenate([acc, jnp.zeros_like(acc)], axis=1)


def _msg(neigh, bfa, kt2):
    blk = 2048
    return pl.pallas_call(
        _msg_body,
        grid=(_E // blk,),
        in_specs=[
            pl.BlockSpec((blk, _UP), lambda i: (i, 0)),
            pl.BlockSpec((blk, 17), lambda i: (i, 0)),
            pl.BlockSpec((_U, 17 * _U), lambda i: (0, 0)),
        ],
        out_specs=pl.BlockSpec((blk, _UP), lambda i: (i, 0)),
        out_shape=jax.ShapeDtypeStruct((_E, _UP), jnp.float32),
    )(neigh, bfa, kt2)


def _gru_body(p0_ref, p1_ref, h_ref, wih_ref, whh_ref, bih_ref, bhh_ref,
              out_ref):
    agg = p0_ref[:, 0:_U] + p1_ref[:, 0:_U]
    hh = h_ref[:, 0:_U]
    gi = jnp.dot(agg, wih_ref[...],
                 preferred_element_type=jnp.float32) + bih_ref[...]
    gh = jnp.dot(hh, whh_ref[...],
                 preferred_element_type=jnp.float32) + bhh_ref[...]
    r = jax.nn.sigmoid(gi[:, 0:_U] + gh[:, 0:_U])
    z = jax.nn.sigmoid(gi[:, _U:2 * _U] + gh[:, _U:2 * _U])
    n = jnp.tanh(gi[:, 2 * _U:3 * _U] + r * gh[:, 2 * _U:3 * _U])
    hnew = (1.0 - z) * n + z * hh
    out_ref[...] = jnp.concatenate([hnew, jnp.zeros_like(hnew)], axis=1)


def _gru(partials, h, wih_t, whh_t, bih2, bhh2):
    blk = 1024
    nb = _N // blk
    return pl.pallas_call(
        _gru_body,
        grid=(nb,),
        in_specs=[
            pl.BlockSpec((blk, _UP), lambda i: (i, 0)),
            pl.BlockSpec((blk, _UP), lambda i, _nb=nb: (i + _nb, 0)),
            pl.BlockSpec((blk, _UP), lambda i: (i, 0)),
            pl.BlockSpec((_U, 3 * _U), lambda i: (0, 0)),
            pl.BlockSpec((_U, 3 * _U), lambda i: (0, 0)),
            pl.BlockSpec((1, 3 * _U), lambda i: (0, 0)),
            pl.BlockSpec((1, 3 * _U), lambda i: (0, 0)),
        ],
        out_specs=pl.BlockSpec((blk, _UP), lambda i: (i, 0)),
        out_shape=jax.ShapeDtypeStruct((_N, _UP), jnp.float32),
    )(partials, partials, h, wih_t, whh_t, bih2, bhh2)


_RT = 256
_PADR = 8


def _attn_body(cnt_ref, x_ref, kb_ref, vc_ref, ipw_ref, ipb_ref, opw_ref,
               opb_ref, ln1g_ref, ln1b_ref, w1_ref, b1_ref, w2_ref, b2_ref,
               ln2g_ref, ln2b_ref, vs_ref, pr_ref):
    hd = _U // 8
    count = cnt_ref[0, 0, 0]
    x = x_ref[0][:, 0:_U] * vc_ref[0]
    kb = kb_ref[0]
    qkv = jnp.dot(x.astype(jnp.bfloat16), ipw_ref[...],
                  preferred_element_type=jnp.float32) + ipb_ref[...]
    scale = 1.0 / jnp.sqrt(jnp.float32(hd))
    q = (qkv[:, 0:_U] * scale).astype(jnp.bfloat16)
    k = qkv[:, _U:2 * _U].astype(jnp.bfloat16)
    v = qkv[:, 2 * _U:3 * _U].astype(jnp.bfloat16)

    def tile(r0, rows):
        qt = q[r0:r0 + rows, :]
        ctx_parts = []
        for h in range(8):
            qh = qt[:, h * hd:(h + 1) * hd]
            kh = k[:, h * hd:(h + 1) * hd]
            vh = v[:, h * hd:(h + 1) * hd]
            s = lax.dot_general(qh, kh, (((1,), (1,)), ((), ())),
                                preferred_element_type=jnp.float32) + kb
            m = jnp.max(s, axis=1, keepdims=True)
            e = jnp.exp(s - m)
            a = (e / jnp.sum(e, axis=1, keepdims=True)).astype(jnp.bfloat16)
            ctx_parts.append(
                jnp.dot(a, vh, preferred_element_type=jnp.float32))
        ctx = jnp.concatenate(ctx_parts, axis=1)
        attn_out = jnp.dot(ctx.astype(jnp.bfloat16), opw_ref[...],
                           preferred_element_type=jnp.float32) + opb_ref[...]
        y = x[r0:r0 + rows, :] + attn_out
        mu = jnp.mean(y, axis=1, keepdims=True)
        var = jnp.mean((y - mu) ** 2, axis=1, keepdims=True)
        pin = ((y - mu) / jnp.sqrt(var + 1e-5) * ln1g_ref[...]
               + ln1b_ref[...])
        hid = jnp.maximum(
            jnp.dot(pin.astype(jnp.bfloat16), w1_ref[...],
                    preferred_element_type=jnp.float32)
            + b1_ref[...], 0.0)
        mlp = jnp.dot(hid.astype(jnp.bfloat16), w2_ref[...],
                      preferred_element_type=jnp.float32) + b2_ref[...]
        y2 = pin + mlp
        mu2 = jnp.mean(y2, axis=1, keepdims=True)
        var2 = jnp.mean((y2 - mu2) ** 2, axis=1, keepdims=True)
        return ((y2 - mu2) / jnp.sqrt(var2 + 1e-5) * ln2g_ref[...]
                + ln2b_ref[...])

    vs_ref[0] = jnp.zeros((1, _U), jnp.float32)
    for r in range(_L // _RT):
        @pl.when(count > r * _RT)
        def _():
            pout = tile(r * _RT, _RT)
            vm = vc_ref[0][r * _RT:(r + 1) * _RT, :]
            vs_ref[0] = vs_ref[0] + jnp.sum(pout * vm, axis=0, keepdims=True)
    pout_pad = tile(_L - _PADR, _PADR)
    pr_ref[0] = pout_pad[_PADR - 1:_PADR, :]


def _attn(counts2, x3, kb3, vc, ipw_t, ipb2, opw_t, opb2, ln1g2, ln1b2,
          w1_t, b12, w2_t, b22, ln2g2, ln2b2):
    full = lambda a, b: pl.BlockSpec((a, b), lambda i: (0, 0))
    return pl.pallas_call(
        _attn_body,
        grid=(_B,),
        in_specs=[
            pl.BlockSpec((1, 1, 1), lambda i: (i, 0, 0),
                         memory_space=pltpu.SMEM),
            pl.BlockSpec((1, _L, _UP), lambda i: (i, 0, 0)),
            pl.BlockSpec((1, 1, _L), lambda i: (i, 0, 0)),
            pl.BlockSpec((1, _L, 1), lambda i: (i, 0, 0)),
            full(_U, 3 * _U), full(1, 3 * _U),
            full(_U, _U), full(1, _U),
            full(1, _U), full(1, _U),
            full(_U, _U), full(1, _U),
            full(_U, _U), full(1, _U),
            full(1, _U), full(1, _U),
        ],
        out_specs=(
            pl.BlockSpec((1, 1, _U), lambda i: (i, 0, 0)),
            pl.BlockSpec((1, 1, _U), lambda i: (i, 0, 0)),
        ),
        out_shape=(
            jax.ShapeDtypeStruct((_B, 1, _U), jnp.float32),
            jax.ShapeDtypeStruct((_B, 1, _U), jnp.float32),
        ),
    )(counts2, x3, kb3, vc, ipw_t, ipb2, opw_t, opb2, ln1g2, ln1b2,
      w1_t, b12, w2_t, b22, ln2g2, ln2b2)


def _final_body(vs_ref, pr_ref, cnt_ref, d1_ref, b1_ref, d2_ref, b2_ref,
                out_ref):
    cnt = cnt_ref[...].astype(jnp.float32)
    mx = jnp.max(cnt)
    pooled = (vs_ref[...] + (mx - cnt) * pr_ref[...]) / mx
    hid = jnp.maximum(
        jnp.dot(pooled, d1_ref[...], preferred_element_type=jnp.float32)
        + b1_ref[...], 0.0)
    logit = jnp.dot(hid, d2_ref[...],
                    preferred_element_type=jnp.float32) + b2_ref[...]
    out_ref[...] = jax.nn.sigmoid(logit)


def _final(vs, pr, counts, d1_t, d1b2, d2_t, d2b2):
    return pl.pallas_call(
        _final_body,
        out_shape=jax.ShapeDtypeStruct((_B, 1), jnp.float32),
    )(vs, pr, counts, d1_t, d1b2, d2_t, d2b2)


def kernel(atom_features, bond_features, kernel, bias_p, w_ih, w_hh, b_ih,
           b_hh, ipw, ipb, opw, opb, ln1_g, ln1_b, w1, b1, w2, b2, ln2_g,
           ln2_b, d1_w, d1_b, d2_w, d2_b, pair_indices, molecule_indicator):
    src = pair_indices[:, 1]
    dst2d = pair_indices[:, 0].reshape(_NW * _NCH, _CHUNK)
    bfa = jnp.concatenate(
        [bond_features, jnp.ones((_E, 1), jnp.float32)], axis=1)
    k3 = jnp.concatenate([kernel, bias_p[None, :]], axis=0)
    kt2 = k3.reshape(17, _U, _U).transpose(2, 0, 1).reshape(_U, 17 * _U)
    zeros_nu = jnp.zeros((_N, _UP), jnp.float32)

    gidx, validf, counts, keybias = _plan(molecule_indicator.reshape(1, _N))

    h = jnp.pad(atom_features, ((0, 0), (0, _UP - _U)))
    for _ in range(4):
        neigh = _sc_gather(h, src)
        msg = _msg(neigh, bfa, kt2.astype(jnp.bfloat16))
        partials = _sc_scatter(msg, dst2d, zeros_nu)
        h = _gru(partials, h, w_ih.T, w_hh.T, b_ih[None, :], b_hh[None, :])

    xg = _sc_gather(h, gidx.reshape(_E))
    bf16 = jnp.bfloat16
    vs, pr = _attn(
        counts.reshape(_B, 1, 1), xg.reshape(_B, _L, _UP),
        keybias.reshape(_B, 1, _L),
        validf.reshape(_B, _L, 1),
        ipw.T.astype(bf16), ipb[None, :], opw.T.astype(bf16), opb[None, :],
        ln1_g[None, :], ln1_b[None, :], w1.T.astype(bf16), b1[None, :],
        w2.T.astype(bf16), b2[None, :], ln2_g[None, :], ln2_b[None, :])
    return _final(vs.reshape(_B, _U), pr.reshape(_B, _U), counts, d1_w.T,
                  d1_b[None, :], d2_w.T, d2_b[None, :])


# speedup vs baseline: 3.0514x; 1.2394x over previous
import functools

import jax
import jax.numpy as jnp
from jax import lax
from jax.experimental import pallas as pl
from jax.experimental.pallas import tpu as pltpu
from jax.experimental.pallas import tpu_sc as plsc

_E = 16384
_N = 4096
_U = 64
_B = 32
_L = 512
_NC, _NS = 2, 16
_NW = _NC * _NS
_EPW = _E // _NW
_CHUNK = 128
_NCH = _EPW // _CHUNK
_RPS = _N // _NS

_UP = 128
_SC_MESH = plsc.VectorSubcoreMesh(core_axis_name="c", subcore_axis_name="s")


def _gather_body(table_hbm, idx_hbm, out_hbm, idx_v, rows_v, sem):
    wid = lax.axis_index("s") * _NC + lax.axis_index("c")
    base = wid * _EPW
    pltpu.sync_copy(idx_hbm.at[pl.ds(base, _EPW)], idx_v)
    descs = [
        pltpu.async_copy(
            table_hbm.at[idx_v.at[pl.ds(j * _CHUNK, _CHUNK)]],
            rows_v.at[pl.ds(j * _CHUNK, _CHUNK)], sem)
        for j in range(_NCH)
    ]
    for d in descs:
        d.wait()
    pltpu.sync_copy(rows_v, out_hbm.at[pl.ds(base, _EPW)])


def _sc_gather(table, idx):
    return pl.kernel(
        _gather_body,
        out_type=jax.ShapeDtypeStruct((_E, _UP), jnp.float32),
        mesh=_SC_MESH,
        scratch_types=[
            pltpu.VMEM((_EPW,), jnp.int32),
            pltpu.VMEM((_EPW, _UP), jnp.float32),
            pltpu.SemaphoreType.DMA,
        ],
    )(table, idx)


def _scatter_body(msg_hbm, dst_hbm, zero_hbm, out_hbm, idx_v, msg_v, agg_sh):
    cid = lax.axis_index("c")
    sid = lax.axis_index("s")
    wid = sid * _NC + cid
    rows0 = sid * _RPS
    pltpu.sync_copy(zero_hbm.at[pl.ds(rows0, _RPS)],
                    agg_sh.at[pl.ds(rows0, _RPS)])
    plsc.subcore_barrier()
    base = wid * _EPW
    pltpu.sync_copy(msg_hbm.at[pl.ds(base, _EPW)], msg_v)
    pltpu.sync_copy(dst_hbm.at[pl.ds(wid * _NCH, _NCH)], idx_v)
    for j in range(_NCH):
        pltpu.sync_copy(msg_v.at[pl.ds(j * _CHUNK, _CHUNK)],
                        agg_sh.at[idx_v.at[j]], add=True)
    plsc.subcore_barrier()
    pltpu.sync_copy(agg_sh.at[pl.ds(rows0, _RPS)],
                    out_hbm.at[pl.ds(cid * _N + rows0, _RPS)])


def _sc_scatter(msg, dst2d, zeros_nu):
    return pl.kernel(
        _scatter_body,
        out_type=jax.ShapeDtypeStruct((_NC * _N, _UP), jnp.float32),
        mesh=_SC_MESH,
        scratch_types=[
            pltpu.VMEM((_NCH, _CHUNK), jnp.int32),
            pltpu.VMEM((_EPW, _UP), jnp.float32),
            pltpu.VMEM_SHARED((_N, _UP), jnp.float32),
        ],
    )(msg, dst2d, zeros_nu)


def _plan_body(mol_ref, gidx_ref, valid_ref, counts_ref, kb_ref):
    mol = mol_ref[...]
    mids = lax.broadcasted_iota(jnp.int32, (_B, _N), 0)
    counts = jnp.sum((mol == mids).astype(jnp.int32), axis=1, keepdims=True)
    starts = jnp.sum((mids > mol).astype(jnp.int32), axis=1, keepdims=True)
    p = lax.broadcasted_iota(jnp.int32, (_B, _L), 1)
    mrow = lax.broadcasted_iota(jnp.int32, (_B, _L), 0)
    lim = jnp.minimum(counts, _L)
    valid = (p < counts).astype(jnp.float32)
    valid_ref[...] = valid
    gidx_ref[...] = jnp.where(p < lim, starts + p,
                              jnp.bitwise_and(mrow * _L + p, _N - 1))
    counts_ref[...] = counts
    kb_ref[...] = (valid - 1.0) * jnp.float32(1e9)


def _plan(mol2d):
    return pl.pallas_call(
        _plan_body,
        out_shape=(
            jax.ShapeDtypeStruct((_B, _L), jnp.int32),
            jax.ShapeDtypeStruct((_B, _L), jnp.float32),
            jax.ShapeDtypeStruct((_B, 1), jnp.int32),
            jax.ShapeDtypeStruct((_B, _L), jnp.float32),
        ),
    )(mol2d)


def _msg_body(neigh_ref, bfa_ref, kt_ref, out_ref):
    nb = neigh_ref[:, 0:_U].astype(jnp.bfloat16)
    p = jnp.dot(nb, kt_ref[...],
                preferred_element_type=jnp.float32)
    acc = bfa_ref[:, 0:1] * p[:, 0:_U]
    for b in range(1, 17):
        acc = acc + bfa_ref[:, b:b + 1] * p[:, b * _U:(b + 1) * _U]
    out_ref[...] = jnp.concatenate([acc, jnp.zeros_like(acc)], axis=1)


def _msg(neigh, bfa, kt2):
    blk = 2048
    return pl.pallas_call(
        _msg_body,
        grid=(_E // blk,),
        in_specs=[
            pl.BlockSpec((blk, _UP), lambda i: (i, 0)),
            pl.BlockSpec((blk, 17), lambda i: (i, 0)),
            pl.BlockSpec((_U, 17 * _U), lambda i: (0, 0)),
        ],
        out_specs=pl.BlockSpec((blk, _UP), lambda i: (i, 0)),
        out_shape=jax.ShapeDtypeStruct((_E, _UP), jnp.float32),
    )(neigh, bfa, kt2)


def _gru_body(p0_ref, p1_ref, h_ref, wih_ref, whh_ref, bih_ref, bhh_ref,
              out_ref):
    agg = p0_ref[:, 0:_U] + p1_ref[:, 0:_U]
    hh = h_ref[:, 0:_U]
    gi = jnp.dot(agg, wih_ref[...],
                 preferred_element_type=jnp.float32) + bih_ref[...]
    gh = jnp.dot(hh, whh_ref[...],
                 preferred_element_type=jnp.float32) + bhh_ref[...]
    r = jax.nn.sigmoid(gi[:, 0:_U] + gh[:, 0:_U])
    z = jax.nn.sigmoid(gi[:, _U:2 * _U] + gh[:, _U:2 * _U])
    n = jnp.tanh(gi[:, 2 * _U:3 * _U] + r * gh[:, 2 * _U:3 * _U])
    hnew = (1.0 - z) * n + z * hh
    out_ref[...] = jnp.concatenate([hnew, jnp.zeros_like(hnew)], axis=1)


def _gru(partials, h, wih_t, whh_t, bih2, bhh2):
    blk = 1024
    nb = _N // blk
    return pl.pallas_call(
        _gru_body,
        grid=(nb,),
        in_specs=[
            pl.BlockSpec((blk, _UP), lambda i: (i, 0)),
            pl.BlockSpec((blk, _UP), lambda i, _nb=nb: (i + _nb, 0)),
            pl.BlockSpec((blk, _UP), lambda i: (i, 0)),
            pl.BlockSpec((_U, 3 * _U), lambda i: (0, 0)),
            pl.BlockSpec((_U, 3 * _U), lambda i: (0, 0)),
            pl.BlockSpec((1, 3 * _U), lambda i: (0, 0)),
            pl.BlockSpec((1, 3 * _U), lambda i: (0, 0)),
        ],
        out_specs=pl.BlockSpec((blk, _UP), lambda i: (i, 0)),
        out_shape=jax.ShapeDtypeStruct((_N, _UP), jnp.float32),
    )(partials, partials, h, wih_t, whh_t, bih2, bhh2)


_RT = 256
_PADR = 8


def _attn_body(cnt_ref, x_ref, kb_ref, vc_ref, ipw_ref, ipb_ref, opw_ref,
               opb_ref, ln1g_ref, ln1b_ref, w1_ref, b1_ref, w2_ref, b2_ref,
               ln2g_ref, ln2b_ref, vs_ref, pr_ref):
    hd = _U // 8
    count = cnt_ref[0, 0, 0]
    x = x_ref[0][:, 0:_U] * vc_ref[0]
    kb = kb_ref[0]
    qkv = jnp.dot(x.astype(jnp.bfloat16), ipw_ref[...],
                  preferred_element_type=jnp.float32) + ipb_ref[...]
    scale = 1.0 / jnp.sqrt(jnp.float32(hd))
    q = (qkv[:, 0:_U] * scale).astype(jnp.bfloat16)
    k = qkv[:, _U:2 * _U].astype(jnp.bfloat16)
    v = qkv[:, 2 * _U:3 * _U]

    def tile(r0, rows):
        qt = q[r0:r0 + rows, :]
        ss = [lax.dot_general(qt[:, h * hd:(h + 1) * hd],
                              k[:, h * hd:(h + 1) * hd],
                              (((1,), (1,)), ((), ())),
                              preferred_element_type=jnp.float32) + kb
              for h in range(8)]
        ms = [jnp.max(s, axis=1, keepdims=True) for s in ss]
        es = [jnp.exp(s - m) for s, m in zip(ss, ms)]
        rs = [1.0 / jnp.sum(e, axis=1, keepdims=True) for e in es]
        ctx_parts = [
            jnp.dot(e, v[:, h * hd:(h + 1) * hd],
                    preferred_element_type=jnp.float32) * r
            for h, (e, r) in enumerate(zip(es, rs))]
        ctx = jnp.concatenate(ctx_parts, axis=1)
        attn_out = jnp.dot(ctx.astype(jnp.bfloat16), opw_ref[...],
                           preferred_element_type=jnp.float32) + opb_ref[...]
        y = x[r0:r0 + rows, :] + attn_out
        mu = jnp.mean(y, axis=1, keepdims=True)
        var = jnp.mean((y - mu) ** 2, axis=1, keepdims=True)
        pin = ((y - mu) / jnp.sqrt(var + 1e-5) * ln1g_ref[...]
               + ln1b_ref[...])
        hid = jnp.maximum(
            jnp.dot(pin.astype(jnp.bfloat16), w1_ref[...],
                    preferred_element_type=jnp.float32)
            + b1_ref[...], 0.0)
        mlp = jnp.dot(hid.astype(jnp.bfloat16), w2_ref[...],
                      preferred_element_type=jnp.float32) + b2_ref[...]
        y2 = pin + mlp
        mu2 = jnp.mean(y2, axis=1, keepdims=True)
        var2 = jnp.mean((y2 - mu2) ** 2, axis=1, keepdims=True)
        return ((y2 - mu2) / jnp.sqrt(var2 + 1e-5) * ln2g_ref[...]
                + ln2b_ref[...])

    vs_ref[0] = jnp.zeros((1, _U), jnp.float32)
    for r in range(_L // _RT):
        @pl.when(count > r * _RT)
        def _():
            pout = tile(r * _RT, _RT)
            vm = vc_ref[0][r * _RT:(r + 1) * _RT, :]
            vs_ref[0] = vs_ref[0] + jnp.sum(pout * vm, axis=0, keepdims=True)
    pout_pad = tile(_L - _PADR, _PADR)
    pr_ref[0] = pout_pad[_PADR - 1:_PADR, :]


def _attn(counts2, x3, kb3, vc, ipw_t, ipb2, opw_t, opb2, ln1g2, ln1b2,
          w1_t, b12, w2_t, b22, ln2g2, ln2b2):
    full = lambda a, b: pl.BlockSpec((a, b), lambda i: (0, 0))
    return pl.pallas_call(
        _attn_body,
        grid=(_B,),
        in_specs=[
            pl.BlockSpec((1, 1, 1), lambda i: (i, 0, 0),
                         memory_space=pltpu.SMEM),
            pl.BlockSpec((1, _L, _UP), lambda i: (i, 0, 0)),
            pl.BlockSpec((1, 1, _L), lambda i: (i, 0, 0)),
            pl.BlockSpec((1, _L, 1), lambda i: (i, 0, 0)),
            full(_U, 3 * _U), full(1, 3 * _U),
            full(_U, _U), full(1, _U),
            full(1, _U), full(1, _U),
            full(_U, _U), full(1, _U),
            full(_U, _U), full(1, _U),
            full(1, _U), full(1, _U),
        ],
        out_specs=(
            pl.BlockSpec((1, 1, _U), lambda i: (i, 0, 0)),
            pl.BlockSpec((1, 1, _U), lambda i: (i, 0, 0)),
        ),
        out_shape=(
            jax.ShapeDtypeStruct((_B, 1, _U), jnp.float32),
            jax.ShapeDtypeStruct((_B, 1, _U), jnp.float32),
        ),
    )(counts2, x3, kb3, vc, ipw_t, ipb2, opw_t, opb2, ln1g2, ln1b2,
      w1_t, b12, w2_t, b22, ln2g2, ln2b2)


def _final_body(vs_ref, pr_ref, cnt_ref, d1_ref, b1_ref, d2_ref, b2_ref,
                out_ref):
    cnt = cnt_ref[...].astype(jnp.float32)
    mx = jnp.max(cnt)
    pooled = (vs_ref[...] + (mx - cnt) * pr_ref[...]) / mx
    hid = jnp.maximum(
        jnp.dot(pooled, d1_ref[...], preferred_element_type=jnp.float32)
        + b1_ref[...], 0.0)
    logit = jnp.dot(hid, d2_ref[...],
                    preferred_element_type=jnp.float32) + b2_ref[...]
    out_ref[...] = jax.nn.sigmoid(logit)


def _final(vs, pr, counts, d1_t, d1b2, d2_t, d2b2):
    return pl.pallas_call(
        _final_body,
        out_shape=jax.ShapeDtypeStruct((_B, 1), jnp.float32),
    )(vs, pr, counts, d1_t, d1b2, d2_t, d2b2)


def kernel(atom_features, bond_features, kernel, bias_p, w_ih, w_hh, b_ih,
           b_hh, ipw, ipb, opw, opb, ln1_g, ln1_b, w1, b1, w2, b2, ln2_g,
           ln2_b, d1_w, d1_b, d2_w, d2_b, pair_indices, molecule_indicator):
    src = pair_indices[:, 1]
    dst2d = pair_indices[:, 0].reshape(_NW * _NCH, _CHUNK)
    bfa = jnp.concatenate(
        [bond_features, jnp.ones((_E, 1), jnp.float32)], axis=1)
    k3 = jnp.concatenate([kernel, bias_p[None, :]], axis=0)
    kt2 = k3.reshape(17, _U, _U).transpose(2, 0, 1).reshape(_U, 17 * _U)
    zeros_nu = jnp.zeros((_N, _UP), jnp.float32)

    gidx, validf, counts, keybias = _plan(molecule_indicator.reshape(1, _N))

    h = jnp.pad(atom_features, ((0, 0), (0, _UP - _U)))
    for _ in range(4):
        neigh = _sc_gather(h, src)
        msg = _msg(neigh, bfa, kt2.astype(jnp.bfloat16))
        partials = _sc_scatter(msg, dst2d, zeros_nu)
        h = _gru(partials, h, w_ih.T, w_hh.T, b_ih[None, :], b_hh[None, :])

    xg = _sc_gather(h, gidx.reshape(_E))
    bf16 = jnp.bfloat16
    vs, pr = _attn(
        counts.reshape(_B, 1, 1), xg.reshape(_B, _L, _UP),
        keybias.reshape(_B, 1, _L),
        validf.reshape(_B, _L, 1),
        ipw.T.astype(bf16), ipb[None, :], opw.T.astype(bf16), opb[None, :],
        ln1_g[None, :], ln1_b[None, :], w1.T.astype(bf16), b1[None, :],
        w2.T.astype(bf16), b2[None, :], ln2_g[None, :], ln2_b[None, :])
    return _final(vs.reshape(_B, _U), pr.reshape(_B, _U), counts, d1_w.T,
                  d1_b[None, :], d2_w.T, d2_b[None, :])


# speedup vs baseline: 3.1258x; 1.0244x over previous
import functools

import jax
import jax.numpy as jnp
from jax import lax
from jax.experimental import pallas as pl
from jax.experimental.pallas import tpu as pltpu
from jax.experimental.pallas import tpu_sc as plsc

_E = 16384
_N = 4096
_U = 64
_B = 32
_L = 512
_NC, _NS = 2, 16
_NW = _NC * _NS
_EPW = _E // _NW
_CHUNK = 128
_NCH = _EPW // _CHUNK
_RPS = _N // _NS

_UP = 128
_SC_MESH = plsc.VectorSubcoreMesh(core_axis_name="c", subcore_axis_name="s")


def _gather_body(table_hbm, idx_hbm, out_hbm, idx_v, rows_v, sem):
    wid = lax.axis_index("s") * _NC + lax.axis_index("c")
    base = wid * _EPW
    pltpu.sync_copy(idx_hbm.at[pl.ds(base, _EPW)], idx_v)
    descs = [
        pltpu.async_copy(
            table_hbm.at[idx_v.at[pl.ds(j * _CHUNK, _CHUNK)]],
            rows_v.at[pl.ds(j * _CHUNK, _CHUNK)], sem)
        for j in range(_NCH)
    ]
    for d in descs:
        d.wait()
    pltpu.sync_copy(rows_v, out_hbm.at[pl.ds(base, _EPW)])


def _sc_gather(table, idx):
    return pl.kernel(
        _gather_body,
        out_type=jax.ShapeDtypeStruct((_E, _UP), jnp.float32),
        mesh=_SC_MESH,
        scratch_types=[
            pltpu.VMEM((_EPW,), jnp.int32),
            pltpu.VMEM((_EPW, _UP), jnp.float32),
            pltpu.SemaphoreType.DMA,
        ],
    )(table, idx)


def _scatter_body(msg_hbm, dst_hbm, zero_hbm, out_hbm, idx_v, msg_v, agg_sh):
    cid = lax.axis_index("c")
    sid = lax.axis_index("s")
    wid = sid * _NC + cid
    rows0 = sid * _RPS
    pltpu.sync_copy(zero_hbm.at[pl.ds(rows0, _RPS)],
                    agg_sh.at[pl.ds(rows0, _RPS)])
    plsc.subcore_barrier()
    base = wid * _EPW
    pltpu.sync_copy(msg_hbm.at[pl.ds(base, _EPW)], msg_v)
    pltpu.sync_copy(dst_hbm.at[pl.ds(wid * _NCH, _NCH)], idx_v)
    for j in range(_NCH):
        pltpu.sync_copy(msg_v.at[pl.ds(j * _CHUNK, _CHUNK)],
                        agg_sh.at[idx_v.at[j]], add=True)
    plsc.subcore_barrier()
    pltpu.sync_copy(agg_sh.at[pl.ds(rows0, _RPS)],
                    out_hbm.at[pl.ds(cid * _N + rows0, _RPS)])


def _sc_scatter(msg, dst2d, zeros_nu):
    return pl.kernel(
        _scatter_body,
        out_type=jax.ShapeDtypeStruct((_NC * _N, _UP), jnp.float32),
        mesh=_SC_MESH,
        scratch_types=[
            pltpu.VMEM((_NCH, _CHUNK), jnp.int32),
            pltpu.VMEM((_EPW, _UP), jnp.float32),
            pltpu.VMEM_SHARED((_N, _UP), jnp.float32),
        ],
    )(msg, dst2d, zeros_nu)


def _plan_body(mol_ref, gidx_ref, valid_ref, counts_ref, kb_ref):
    mol = mol_ref[...]
    mids = lax.broadcasted_iota(jnp.int32, (_B, _N), 0)
    counts = jnp.sum((mol == mids).astype(jnp.int32), axis=1, keepdims=True)
    starts = jnp.sum((mids > mol).astype(jnp.int32), axis=1, keepdims=True)
    p = lax.broadcasted_iota(jnp.int32, (_B, _L), 1)
    mrow = lax.broadcasted_iota(jnp.int32, (_B, _L), 0)
    lim = jnp.minimum(counts, _L)
    valid = (p < counts).astype(jnp.float32)
    valid_ref[...] = valid
    gidx_ref[...] = jnp.where(p < lim, starts + p,
                              jnp.bitwise_and(mrow * _L + p, _N - 1))
    counts_ref[...] = counts
    kb_ref[...] = (valid - 1.0) * jnp.float32(1e9)


def _plan(mol2d):
    return pl.pallas_call(
        _plan_body,
        out_shape=(
            jax.ShapeDtypeStruct((_B, _L), jnp.int32),
            jax.ShapeDtypeStruct((_B, _L), jnp.float32),
            jax.ShapeDtypeStruct((_B, 1), jnp.int32),
            jax.ShapeDtypeStruct((_B, _L), jnp.float32),
        ),
    )(mol2d)


def _msg_body(neigh_ref, bfa_ref, kt_ref, out_ref):
    nb = neigh_ref[:, 0:_U].astype(jnp.bfloat16)
    p = jnp.dot(nb, kt_ref[...],
                preferred_element_type=jnp.float32)
    acc = bfa_ref[:, 0:1] * p[:, 0:_U]
    for b in range(1, 17):
        acc = acc + bfa_ref[:, b:b + 1] * p[:, b * _U:(b + 1) * _U]
    out_ref[...] = jnp.concatenate([acc, jnp.zeros_like(acc)], axis=1)


def _msg(neigh, bfa, kt2):
    blk = 2048
    return pl.pallas_call(
        _msg_body,
        grid=(_E // blk,),
        in_specs=[
            pl.BlockSpec((blk, _UP), lambda i: (i, 0)),
            pl.BlockSpec((blk, 17), lambda i: (i, 0)),
            pl.BlockSpec((_U, 17 * _U), lambda i: (0, 0)),
        ],
        out_specs=pl.BlockSpec((blk, _UP), lambda i: (i, 0)),
        out_shape=jax.ShapeDtypeStruct((_E, _UP), jnp.float32),
    )(neigh, bfa, kt2)


def _gru_body(p0_ref, p1_ref, h_ref, wih_ref, whh_ref, bih_ref, bhh_ref,
              out_ref):
    agg = p0_ref[:, 0:_U] + p1_ref[:, 0:_U]
    hh = h_ref[:, 0:_U]
    gi = jnp.dot(agg, wih_ref[...],
                 preferred_element_type=jnp.float32) + bih_ref[...]
    gh = jnp.dot(hh, whh_ref[...],
                 preferred_element_type=jnp.float32) + bhh_ref[...]
    r = jax.nn.sigmoid(gi[:, 0:_U] + gh[:, 0:_U])
    z = jax.nn.sigmoid(gi[:, _U:2 * _U] + gh[:, _U:2 * _U])
    n = jnp.tanh(gi[:, 2 * _U:3 * _U] + r * gh[:, 2 * _U:3 * _U])
    hnew = (1.0 - z) * n + z * hh
    out_ref[...] = jnp.concatenate([hnew, jnp.zeros_like(hnew)], axis=1)


def _gru(partials, h, wih_t, whh_t, bih2, bhh2):
    blk = 1024
    nb = _N // blk
    return pl.pallas_call(
        _gru_body,
        grid=(nb,),
        in_specs=[
            pl.BlockSpec((blk, _UP), lambda i: (i, 0)),
            pl.BlockSpec((blk, _UP), lambda i, _nb=nb: (i + _nb, 0)),
            pl.BlockSpec((blk, _UP), lambda i: (i, 0)),
            pl.BlockSpec((_U, 3 * _U), lambda i: (0, 0)),
            pl.BlockSpec((_U, 3 * _U), lambda i: (0, 0)),
            pl.BlockSpec((1, 3 * _U), lambda i: (0, 0)),
            pl.BlockSpec((1, 3 * _U), lambda i: (0, 0)),
        ],
        out_specs=pl.BlockSpec((blk, _UP), lambda i: (i, 0)),
        out_shape=jax.ShapeDtypeStruct((_N, _UP), jnp.float32),
    )(partials, partials, h, wih_t, whh_t, bih2, bhh2)


_RT = 256
_PADR = 8


def _attn_body(cnt_ref, x_ref, kb_ref, vc_ref, ipw_ref, ipb_ref, opw_ref,
               opb_ref, ln1g_ref, ln1b_ref, w1_ref, b1_ref, w2_ref, b2_ref,
               ln2g_ref, ln2b_ref, vs_ref, pr_ref):
    hd = _U // 8
    count = cnt_ref[0, 0, 0]
    x = x_ref[0][:, 0:_U] * vc_ref[0]
    kb = kb_ref[0]
    qkv = jnp.dot(x.astype(jnp.bfloat16), ipw_ref[...],
                  preferred_element_type=jnp.float32) + ipb_ref[...]
    scale = 1.0 / jnp.sqrt(jnp.float32(hd))
    q = (qkv[:, 0:_U] * scale).astype(jnp.bfloat16)
    k = qkv[:, _U:2 * _U].astype(jnp.bfloat16)
    v = qkv[:, 2 * _U:3 * _U]

    def tile(r0, rows, nk):
        qt = q[r0:r0 + rows, :]
        kbn = kb[:, 0:nk]
        ss = [lax.dot_general(qt[:, h * hd:(h + 1) * hd],
                              k[0:nk, h * hd:(h + 1) * hd],
                              (((1,), (1,)), ((), ())),
                              preferred_element_type=jnp.float32) + kbn
              for h in range(8)]
        ms = [jnp.max(s, axis=1, keepdims=True) for s in ss]
        es = [jnp.exp(s - m) for s, m in zip(ss, ms)]
        rs = [1.0 / jnp.sum(e, axis=1, keepdims=True) for e in es]
        ctx_parts = [
            jnp.dot(e, v[0:nk, h * hd:(h + 1) * hd],
                    preferred_element_type=jnp.float32) * r
            for h, (e, r) in enumerate(zip(es, rs))]
        ctx = jnp.concatenate(ctx_parts, axis=1)
        attn_out = jnp.dot(ctx.astype(jnp.bfloat16), opw_ref[...],
                           preferred_element_type=jnp.float32) + opb_ref[...]
        y = x[r0:r0 + rows, :] + attn_out
        mu = jnp.mean(y, axis=1, keepdims=True)
        var = jnp.mean((y - mu) ** 2, axis=1, keepdims=True)
        pin = ((y - mu) / jnp.sqrt(var + 1e-5) * ln1g_ref[...]
               + ln1b_ref[...])
        hid = jnp.maximum(
            jnp.dot(pin.astype(jnp.bfloat16), w1_ref[...],
                    preferred_element_type=jnp.float32)
            + b1_ref[...], 0.0)
        mlp = jnp.dot(hid.astype(jnp.bfloat16), w2_ref[...],
                      preferred_element_type=jnp.float32) + b2_ref[...]
        y2 = pin + mlp
        mu2 = jnp.mean(y2, axis=1, keepdims=True)
        var2 = jnp.mean((y2 - mu2) ** 2, axis=1, keepdims=True)
        return ((y2 - mu2) / jnp.sqrt(var2 + 1e-5) * ln2g_ref[...]
                + ln2b_ref[...])

    vs_ref[0] = jnp.zeros((1, _U), jnp.float32)

    @pl.when(count <= _RT)
    def _():
        pout = tile(0, _RT, _RT)
        vm = vc_ref[0][0:_RT, :]
        vs_ref[0] = vs_ref[0] + jnp.sum(pout * vm, axis=0, keepdims=True)
        pout_pad = tile(_L - _PADR, _PADR, _RT)
        pr_ref[0] = pout_pad[_PADR - 1:_PADR, :]

    @pl.when(count > _RT)
    def _():
        for r in range(_L // _RT):
            @pl.when(count > r * _RT)
            def _():
                pout = tile(r * _RT, _RT, _L)
                vm = vc_ref[0][r * _RT:(r + 1) * _RT, :]
                vs_ref[0] = (vs_ref[0]
                             + jnp.sum(pout * vm, axis=0, keepdims=True))
        pout_pad = tile(_L - _PADR, _PADR, _L)
        pr_ref[0] = pout_pad[_PADR - 1:_PADR, :]


def _attn(counts2, x3, kb3, vc, ipw_t, ipb2, opw_t, opb2, ln1g2, ln1b2,
          w1_t, b12, w2_t, b22, ln2g2, ln2b2):
    full = lambda a, b: pl.BlockSpec((a, b), lambda i: (0, 0))
    return pl.pallas_call(
        _attn_body,
        grid=(_B,),
        in_specs=[
            pl.BlockSpec((1, 1, 1), lambda i: (i, 0, 0),
                         memory_space=pltpu.SMEM),
            pl.BlockSpec((1, _L, _UP), lambda i: (i, 0, 0)),
            pl.BlockSpec((1, 1, _L), lambda i: (i, 0, 0)),
            pl.BlockSpec((1, _L, 1), lambda i: (i, 0, 0)),
            full(_U, 3 * _U), full(1, 3 * _U),
            full(_U, _U), full(1, _U),
            full(1, _U), full(1, _U),
            full(_U, _U), full(1, _U),
            full(_U, _U), full(1, _U),
            full(1, _U), full(1, _U),
        ],
        out_specs=(
            pl.BlockSpec((1, 1, _U), lambda i: (i, 0, 0)),
            pl.BlockSpec((1, 1, _U), lambda i: (i, 0, 0)),
        ),
        out_shape=(
            jax.ShapeDtypeStruct((_B, 1, _U), jnp.float32),
            jax.ShapeDtypeStruct((_B, 1, _U), jnp.float32),
        ),
    )(counts2, x3, kb3, vc, ipw_t, ipb2, opw_t, opb2, ln1g2, ln1b2,
      w1_t, b12, w2_t, b22, ln2g2, ln2b2)


def _final_body(vs_ref, pr_ref, cnt_ref, d1_ref, b1_ref, d2_ref, b2_ref,
                out_ref):
    cnt = cnt_ref[...].astype(jnp.float32)
    mx = jnp.max(cnt)
    pooled = (vs_ref[...] + (mx - cnt) * pr_ref[...]) / mx
    hid = jnp.maximum(
        jnp.dot(pooled, d1_ref[...], preferred_element_type=jnp.float32)
        + b1_ref[...], 0.0)
    logit = jnp.dot(hid, d2_ref[...],
                    preferred_element_type=jnp.float32) + b2_ref[...]
    out_ref[...] = jax.nn.sigmoid(logit)


def _final(vs, pr, counts, d1_t, d1b2, d2_t, d2b2):
    return pl.pallas_call(
        _final_body,
        out_shape=jax.ShapeDtypeStruct((_B, 1), jnp.float32),
    )(vs, pr, counts, d1_t, d1b2, d2_t, d2b2)


def kernel(atom_features, bond_features, kernel, bias_p, w_ih, w_hh, b_ih,
           b_hh, ipw, ipb, opw, opb, ln1_g, ln1_b, w1, b1, w2, b2, ln2_g,
           ln2_b, d1_w, d1_b, d2_w, d2_b, pair_indices, molecule_indicator):
    src = pair_indices[:, 1]
    dst2d = pair_indices[:, 0].reshape(_NW * _NCH, _CHUNK)
    bfa = jnp.concatenate(
        [bond_features, jnp.ones((_E, 1), jnp.float32)], axis=1)
    k3 = jnp.concatenate([kernel, bias_p[None, :]], axis=0)
    kt2 = k3.reshape(17, _U, _U).transpose(2, 0, 1).reshape(_U, 17 * _U)
    zeros_nu = jnp.zeros((_N, _UP), jnp.float32)

    gidx, validf, counts, keybias = _plan(molecule_indicator.reshape(1, _N))

    h = jnp.pad(atom_features, ((0, 0), (0, _UP - _U)))
    for _ in range(4):
        neigh = _sc_gather(h, src)
        msg = _msg(neigh, bfa, kt2.astype(jnp.bfloat16))
        partials = _sc_scatter(msg, dst2d, zeros_nu)
        h = _gru(partials, h, w_ih.T, w_hh.T, b_ih[None, :], b_hh[None, :])

    xg = _sc_gather(h, gidx.reshape(_E))
    bf16 = jnp.bfloat16
    vs, pr = _attn(
        counts.reshape(_B, 1, 1), xg.reshape(_B, _L, _UP),
        keybias.reshape(_B, 1, _L),
        validf.reshape(_B, _L, 1),
        ipw.T.astype(bf16), ipb[None, :], opw.T.astype(bf16), opb[None, :],
        ln1_g[None, :], ln1_b[None, :], w1.T.astype(bf16), b1[None, :],
        w2.T.astype(bf16), b2[None, :], ln2_g[None, :], ln2_b[None, :])
    return _final(vs.reshape(_B, _U), pr.reshape(_B, _U), counts, d1_w.T,
                  d1_b[None, :], d2_w.T, d2_b[None, :])
